# Initial kernel scaffold; baseline (speedup 1.0000x reference)
import jax
import jax.numpy as jnp
from jax.experimental import pallas as pl


def kernel(x, edge_index, Wl1, Wr1, b1, g1, be1, Wl2, Wr2, b2, g2, be2, Wl3, Wr3, b3):
    raise NotImplementedError("write your pallas kernel here")


# speedup vs baseline: 3.8023x; 3.8023x over previous
import functools

import jax
import jax.numpy as jnp
from jax import lax
from jax.experimental import pallas as pl
from jax.experimental.pallas import tpu as pltpu
from jax.experimental.pallas import tpu_sc as plsc

_ROWS = 1000
_CHUNK = 80
_EPS = 1e-5


def _p_plain_body(n_nodes, h_ref, wl_ref, wr_ref, b_ref, y_ref, z_ref):
    h = h_ref[...]
    y_ref[...] = jnp.dot(h, wl_ref[...], preferred_element_type=jnp.float32)
    z_ref[...] = (jnp.dot(h, wr_ref[...], preferred_element_type=jnp.float32)
                  + b_ref[0:1, :])


def _p_bn_body(n_nodes, h_ref, wl_ref, wr_ref, b_ref, st_ref, g_ref, be_ref,
               y_ref, z_ref):
    m = st_ref[0:1, :] / n_nodes
    var = st_ref[1:2, :] / n_nodes - m * m
    scale = lax.rsqrt(var + _EPS) * g_ref[0:1, :]
    h = jnp.maximum((h_ref[...] - m) * scale + be_ref[0:1, :], 0.0)
    y_ref[...] = jnp.dot(h, wl_ref[...], preferred_element_type=jnp.float32)
    z_ref[...] = (jnp.dot(h, wr_ref[...], preferred_element_type=jnp.float32)
                  + b_ref[0:1, :])


def _prologue(h, wlT, wrT, b, stats=None, g=None, be=None):
    n, d = h.shape
    wo = wlT.shape[1]
    grid = (n // _ROWS,)
    row_spec = pl.BlockSpec((_ROWS, d), lambda i: (i, 0))
    w_spec = pl.BlockSpec((d, wo), lambda i: (0, 0))
    vec_spec = pl.BlockSpec((1, wo), lambda i: (0, 0))
    out_spec = pl.BlockSpec((_ROWS, wo), lambda i: (i, 0))
    out_shape = [jax.ShapeDtypeStruct((n, wo), jnp.float32)] * 2
    if stats is None:
        body = functools.partial(_p_plain_body, n)
        in_specs = [row_spec, w_spec, w_spec, vec_spec]
        args = (h, wlT, wrT, b)
    else:
        body = functools.partial(_p_bn_body, n)
        dvec = pl.BlockSpec((1, d), lambda i: (0, 0))
        in_specs = [row_spec, w_spec, w_spec, vec_spec,
                    pl.BlockSpec((8, d), lambda i: (0, 0)), dvec, dvec]
        args = (h, wlT, wrT, b, stats, g, be)
    return pl.pallas_call(body, grid=grid, in_specs=in_specs,
                          out_specs=[out_spec, out_spec],
                          out_shape=out_shape)(*args)


def _e_stats_body(a0_ref, a1_ref, z_ref, d0_ref, d1_ref, h_ref, st_ref):
    deg = jnp.maximum(d0_ref[...] + d1_ref[...], 1.0)
    h = (a0_ref[...] + a1_ref[...]) / deg + z_ref[...]
    h_ref[...] = h

    @pl.when(pl.program_id(0) == 0)
    def _():
        st_ref[...] = jnp.zeros_like(st_ref)

    st_ref[0:1, :] += jnp.sum(h, axis=0, keepdims=True)
    st_ref[1:2, :] += jnp.sum(h * h, axis=0, keepdims=True)


def _e_plain_body(a0_ref, a1_ref, z_ref, d0_ref, d1_ref, h_ref):
    deg = jnp.maximum(d0_ref[...] + d1_ref[...], 1.0)
    h_ref[...] = (a0_ref[...] + a1_ref[...]) / deg + z_ref[...]


def _epilogue(accs, z, degs, with_stats):
    n, wo = z.shape
    nb = n // _ROWS
    grid = (nb,)
    a0 = pl.BlockSpec((_ROWS, wo), lambda i: (i, 0))
    a1 = pl.BlockSpec((_ROWS, wo), lambda i: (i + nb, 0))
    d0 = pl.BlockSpec((_ROWS, 1), lambda i: (i, 0))
    d1 = pl.BlockSpec((_ROWS, 1), lambda i: (i + nb, 0))
    zs = pl.BlockSpec((_ROWS, wo), lambda i: (i, 0))
    out_shape = [jax.ShapeDtypeStruct((n, wo), jnp.float32)]
    out_specs = [pl.BlockSpec((_ROWS, wo), lambda i: (i, 0))]
    body = _e_stats_body if with_stats else _e_plain_body
    if with_stats:
        out_shape.append(jax.ShapeDtypeStruct((8, 128), jnp.float32))
        out_specs.append(pl.BlockSpec((8, 128), lambda i: (0, 0)))
    return pl.pallas_call(body, grid=grid,
                          in_specs=[a0, a1, zs, d0, d1],
                          out_specs=out_specs,
                          out_shape=out_shape)(accs, accs, z, degs, degs)


def _make_sc_agg(n_nodes, width, e_total, with_deg):
    nw = 32
    epw = e_total // nw
    nch = epw // _CHUNK
    rpt = (n_nodes // 16) // 8 * 8
    tail = n_nodes - 16 * rpt
    mesh = plsc.VectorSubcoreMesh(core_axis_name="c", subcore_axis_name="s")

    out_type = [jax.ShapeDtypeStruct((2 * n_nodes, width), jnp.float32)]
    scratch = [
        pltpu.VMEM((_CHUNK,), jnp.int32),
        pltpu.VMEM((_CHUNK,), jnp.int32),
        pltpu.VMEM((_CHUNK, width), jnp.float32),
        pltpu.VMEM_SHARED((n_nodes, width), jnp.float32),
        pltpu.SemaphoreType.DMA,
    ]
    if with_deg:
        out_type.append(jax.ShapeDtypeStruct((2 * n_nodes, 16), jnp.float32))
        scratch += [
            pltpu.VMEM((_CHUNK, 16), jnp.float32),
            pltpu.VMEM_SHARED((n_nodes, 16), jnp.float32),
        ]

    def body(*refs):
        if with_deg:
            (y_hbm, src_hbm, dst_hbm, zf_hbm, z16_hbm, ones_hbm,
             acc_out, deg_out, srcc, dstc, rows_v, acc_sh, sem,
             ones_v, deg_sh) = refs
        else:
            (y_hbm, src_hbm, dst_hbm, zf_hbm,
             acc_out, srcc, dstc, rows_v, acc_sh, sem) = refs
        c = lax.axis_index("c")
        s = lax.axis_index("s")
        w = c * 16 + s
        r0 = s * rpt
        t0 = 16 * rpt
        pltpu.sync_copy(zf_hbm.at[pl.ds(r0, rpt)], acc_sh.at[pl.ds(r0, rpt)])
        if with_deg:
            pltpu.sync_copy(z16_hbm.at[pl.ds(r0, rpt)],
                            deg_sh.at[pl.ds(r0, rpt)])
            pltpu.sync_copy(ones_hbm, ones_v)
        if tail:
            @pl.when(s == 15)
            def _():
                pltpu.sync_copy(zf_hbm.at[pl.ds(t0, tail)],
                                acc_sh.at[pl.ds(t0, tail)])
                if with_deg:
                    pltpu.sync_copy(z16_hbm.at[pl.ds(t0, tail)],
                                    deg_sh.at[pl.ds(t0, tail)])
        plsc.subcore_barrier()

        def step(j, carry):
            base = w * epw + j * _CHUNK
            pltpu.sync_copy(src_hbm.at[pl.ds(base, _CHUNK)], srcc)
            pltpu.sync_copy(dst_hbm.at[pl.ds(base, _CHUNK)], dstc)
            pltpu.async_copy(y_hbm.at[srcc], rows_v, sem).wait()
            pltpu.sync_copy(rows_v, acc_sh.at[dstc], add=True)
            if with_deg:
                pltpu.sync_copy(ones_v, deg_sh.at[dstc], add=True)
            return carry

        lax.fori_loop(0, nch, step, 0)
        plsc.subcore_barrier()
        o0 = c * n_nodes + r0
        pltpu.sync_copy(acc_sh.at[pl.ds(r0, rpt)], acc_out.at[pl.ds(o0, rpt)])
        if with_deg:
            pltpu.sync_copy(deg_sh.at[pl.ds(r0, rpt)],
                            deg_out.at[pl.ds(o0, rpt)])
        if tail:
            @pl.when(s == 15)
            def _():
                ot = c * n_nodes + t0
                pltpu.sync_copy(acc_sh.at[pl.ds(t0, tail)],
                                acc_out.at[pl.ds(ot, tail)])
                if with_deg:
                    pltpu.sync_copy(deg_sh.at[pl.ds(t0, tail)],
                                    deg_out.at[pl.ds(ot, tail)])

    return pl.kernel(body, out_type=tuple(out_type), mesh=mesh,
                     scratch_types=tuple(scratch))


def kernel(x, edge_index, Wl1, Wr1, b1, g1, be1, Wl2, Wr2, b2, g2, be2,
           Wl3, Wr3, b3):
    n, d = x.shape
    e = edge_index.shape[1]
    assert e % (32 * _CHUNK) == 0 and n % 16 == 0 and n % _ROWS == 0

    src = edge_index[0]
    dst = edge_index[1]
    zf = jnp.zeros((n, d), jnp.float32)

    agg = _make_sc_agg(n, d, e, False)

    deg = jax.ops.segment_sum(jnp.ones((e,), jnp.float32), dst, num_segments=n)
    degs = jnp.concatenate([deg, jnp.zeros_like(deg)]).reshape(2 * n, 1)

    y1, zz1 = _prologue(x, Wl1.T, Wr1.T, b1.reshape(1, -1))
    (acc1,) = agg(y1, src, dst, zf)
    h1, st1 = _epilogue(acc1, zz1, degs, True)

    y2, zz2 = _prologue(h1, Wl2.T, Wr2.T, b2.reshape(1, -1),
                        st1, g1.reshape(1, -1), be1.reshape(1, -1))
    (acc2,) = agg(y2, src, dst, zf)
    h2, st2 = _epilogue(acc2, zz2, degs, True)

    w3l = jnp.pad(Wl3.T, ((0, 0), (0, d - 1)))
    w3r = jnp.pad(Wr3.T, ((0, 0), (0, d - 1)))
    b3w = jnp.pad(b3.reshape(1, 1), ((0, 0), (0, d - 1)))
    y3, zz3 = _prologue(h2, w3l, w3r, b3w,
                        st2, g2.reshape(1, -1), be2.reshape(1, -1))
    (acc3,) = agg(y3, src, dst, zf)
    (out_w,) = _epilogue(acc3, zz3, degs, False)
    return out_w[:, 0:1]


# speedup vs baseline: 4.6623x; 1.2262x over previous
import functools

import jax
import jax.numpy as jnp
from jax import lax
from jax.experimental import pallas as pl
from jax.experimental.pallas import tpu as pltpu
from jax.experimental.pallas import tpu_sc as plsc

_ROWS = 1000
_CHUNK = 80
_EPS = 1e-5


def _p_plain_body(n_nodes, h_ref, wl_ref, wr_ref, b_ref, y_ref, z_ref):
    h = h_ref[...]
    y_ref[...] = jnp.dot(h, wl_ref[...], preferred_element_type=jnp.float32)
    z_ref[...] = (jnp.dot(h, wr_ref[...], preferred_element_type=jnp.float32)
                  + b_ref[0:1, :])


def _p_bn_body(n_nodes, h_ref, wl_ref, wr_ref, b_ref, st_ref, g_ref, be_ref,
               y_ref, z_ref):
    m = st_ref[0:1, :] / n_nodes
    var = st_ref[1:2, :] / n_nodes - m * m
    scale = lax.rsqrt(var + _EPS) * g_ref[0:1, :]
    h = jnp.maximum((h_ref[...] - m) * scale + be_ref[0:1, :], 0.0)
    y_ref[...] = jnp.dot(h, wl_ref[...], preferred_element_type=jnp.float32)
    z_ref[...] = (jnp.dot(h, wr_ref[...], preferred_element_type=jnp.float32)
                  + b_ref[0:1, :])


def _prologue(h, wlT, wrT, b, stats=None, g=None, be=None):
    n, d = h.shape
    wo = wlT.shape[1]
    grid = (n // _ROWS,)
    row_spec = pl.BlockSpec((_ROWS, d), lambda i: (i, 0))
    w_spec = pl.BlockSpec((d, wo), lambda i: (0, 0))
    vec_spec = pl.BlockSpec((1, wo), lambda i: (0, 0))
    out_spec = pl.BlockSpec((_ROWS, wo), lambda i: (i, 0))
    out_shape = [jax.ShapeDtypeStruct((n, wo), jnp.float32)] * 2
    if stats is None:
        body = functools.partial(_p_plain_body, n)
        in_specs = [row_spec, w_spec, w_spec, vec_spec]
        args = (h, wlT, wrT, b)
    else:
        body = functools.partial(_p_bn_body, n)
        dvec = pl.BlockSpec((1, d), lambda i: (0, 0))
        in_specs = [row_spec, w_spec, w_spec, vec_spec,
                    pl.BlockSpec((8, d), lambda i: (0, 0)), dvec, dvec]
        args = (h, wlT, wrT, b, stats, g, be)
    return pl.pallas_call(body, grid=grid, in_specs=in_specs,
                          out_specs=[out_spec, out_spec],
                          out_shape=out_shape)(*args)


def _e_stats_body(a0_ref, a1_ref, z_ref, d0_ref, d1_ref, h_ref, st_ref):
    deg = jnp.maximum(d0_ref[:, 0:1] + d1_ref[:, 0:1], 1.0)
    h = (a0_ref[...] + a1_ref[...]) / deg + z_ref[...]
    h_ref[...] = h

    @pl.when(pl.program_id(0) == 0)
    def _():
        st_ref[...] = jnp.zeros_like(st_ref)

    st_ref[0:1, :] += jnp.sum(h, axis=0, keepdims=True)
    st_ref[1:2, :] += jnp.sum(h * h, axis=0, keepdims=True)


def _e_plain_body(a0_ref, a1_ref, z_ref, d0_ref, d1_ref, h_ref):
    deg = jnp.maximum(d0_ref[:, 0:1] + d1_ref[:, 0:1], 1.0)
    h_ref[...] = (a0_ref[...] + a1_ref[...]) / deg + z_ref[...]


def _epilogue(accs, z, degs, with_stats):
    n, wo = z.shape
    nb = n // _ROWS
    grid = (nb,)
    a0 = pl.BlockSpec((_ROWS, wo), lambda i: (i, 0))
    a1 = pl.BlockSpec((_ROWS, wo), lambda i: (i + nb, 0))
    d0 = pl.BlockSpec((_ROWS, 128), lambda i: (i, 0))
    d1 = pl.BlockSpec((_ROWS, 128), lambda i: (i + nb, 0))
    zs = pl.BlockSpec((_ROWS, wo), lambda i: (i, 0))
    out_shape = [jax.ShapeDtypeStruct((n, wo), jnp.float32)]
    out_specs = [pl.BlockSpec((_ROWS, wo), lambda i: (i, 0))]
    body = _e_stats_body if with_stats else _e_plain_body
    if with_stats:
        out_shape.append(jax.ShapeDtypeStruct((8, 128), jnp.float32))
        out_specs.append(pl.BlockSpec((8, 128), lambda i: (0, 0)))
    return pl.pallas_call(body, grid=grid,
                          in_specs=[a0, a1, zs, d0, d1],
                          out_specs=out_specs,
                          out_shape=out_shape)(accs, accs, z, degs, degs)


def _make_sc_agg(n_nodes, width, e_total, ones_mode):
    nw = 32
    epw = e_total // nw
    nch = epw // _CHUNK
    rpt = (n_nodes // 16) // 8 * 8
    tail = n_nodes - 16 * rpt
    mesh = plsc.VectorSubcoreMesh(core_axis_name="c", subcore_axis_name="s")

    out_type = [jax.ShapeDtypeStruct((2 * n_nodes, width), jnp.float32)]
    scratch = [
        pltpu.VMEM((_CHUNK,), jnp.int32),
        pltpu.VMEM((_CHUNK,), jnp.int32),
        pltpu.VMEM((_CHUNK, width), jnp.float32),
        pltpu.VMEM_SHARED((n_nodes, width), jnp.float32),
        pltpu.SemaphoreType.DMA,
    ]

    def body(*refs):
        if ones_mode:
            (src_hbm, dst_hbm, zf_hbm,
             acc_out, srcc, dstc, rows_v, acc_sh, sem) = refs
        else:
            (y_hbm, src_hbm, dst_hbm, zf_hbm,
             acc_out, srcc, dstc, rows_v, acc_sh, sem) = refs
        c = lax.axis_index("c")
        s = lax.axis_index("s")
        w = c * 16 + s
        r0 = s * rpt
        t0 = 16 * rpt
        pltpu.sync_copy(zf_hbm.at[pl.ds(r0, rpt)], acc_sh.at[pl.ds(r0, rpt)])
        if tail:
            @pl.when(s == 15)
            def _():
                pltpu.sync_copy(zf_hbm.at[pl.ds(t0, tail)],
                                acc_sh.at[pl.ds(t0, tail)])
        if ones_mode:
            ov = jnp.full((16,), 1.0, jnp.float32)

            def fill_o(j, carry):
                for k in range(width // 16):
                    rows_v[j, pl.ds(k * 16, 16)] = ov
                return carry

            lax.fori_loop(0, _CHUNK, fill_o, 0)
        plsc.subcore_barrier()

        def step(j, carry):
            base = w * epw + j * _CHUNK
            pltpu.sync_copy(dst_hbm.at[pl.ds(base, _CHUNK)], dstc)
            if not ones_mode:
                pltpu.sync_copy(src_hbm.at[pl.ds(base, _CHUNK)], srcc)
                pltpu.async_copy(y_hbm.at[srcc], rows_v, sem).wait()
            pltpu.sync_copy(rows_v, acc_sh.at[dstc], add=True)
            return carry

        lax.fori_loop(0, nch, step, 0)
        plsc.subcore_barrier()
        o0 = c * n_nodes + r0
        pltpu.sync_copy(acc_sh.at[pl.ds(r0, rpt)], acc_out.at[pl.ds(o0, rpt)])
        if tail:
            @pl.when(s == 15)
            def _():
                ot = c * n_nodes + t0
                pltpu.sync_copy(acc_sh.at[pl.ds(t0, tail)],
                                acc_out.at[pl.ds(ot, tail)])

    return pl.kernel(body, out_type=tuple(out_type), mesh=mesh,
                     scratch_types=tuple(scratch))


def kernel(x, edge_index, Wl1, Wr1, b1, g1, be1, Wl2, Wr2, b2, g2, be2,
           Wl3, Wr3, b3):
    n, d = x.shape
    e = edge_index.shape[1]
    assert e % (32 * _CHUNK) == 0 and n % 16 == 0 and n % _ROWS == 0

    src = edge_index[0]
    dst = edge_index[1]
    zf = jnp.zeros((n, d), jnp.float32)

    agg_ones = _make_sc_agg(n, d, e, True)
    agg = _make_sc_agg(n, d, e, False)

    (degs,) = agg_ones(src, dst, zf)

    y1, zz1 = _prologue(x, Wl1.T, Wr1.T, b1.reshape(1, -1))
    (acc1,) = agg(y1, src, dst, zf)
    h1, st1 = _epilogue(acc1, zz1, degs, True)

    y2, zz2 = _prologue(h1, Wl2.T, Wr2.T, b2.reshape(1, -1),
                        st1, g1.reshape(1, -1), be1.reshape(1, -1))
    (acc2,) = agg(y2, src, dst, zf)
    h2, st2 = _epilogue(acc2, zz2, degs, True)

    w3l = jnp.pad(Wl3.T, ((0, 0), (0, d - 1)))
    w3r = jnp.pad(Wr3.T, ((0, 0), (0, d - 1)))
    b3w = jnp.pad(b3.reshape(1, 1), ((0, 0), (0, d - 1)))
    y3, zz3 = _prologue(h2, w3l, w3r, b3w,
                        st2, g2.reshape(1, -1), be2.reshape(1, -1))
    (acc3,) = agg(y3, src, dst, zf)
    (out_w,) = _epilogue(acc3, zz3, degs, False)
    return out_w[:, 0:1]


# speedup vs baseline: 6.9366x; 1.4878x over previous
import functools

import jax
import jax.numpy as jnp
from jax import lax
from jax.experimental import pallas as pl
from jax.experimental.pallas import tpu as pltpu
from jax.experimental.pallas import tpu_sc as plsc

_ROWS = 1000
_CHUNK = 80
_EPS = 1e-5


def _p_plain_body(n_nodes, h_ref, wl_ref, wr_ref, b_ref, y_ref, z_ref):
    h = h_ref[...]
    y_ref[...] = jnp.dot(h, wl_ref[...], preferred_element_type=jnp.float32)
    z_ref[...] = (jnp.dot(h, wr_ref[...], preferred_element_type=jnp.float32)
                  + b_ref[0:1, :])


def _p_bn_body(n_nodes, h_ref, wl_ref, wr_ref, b_ref, st_ref, g_ref, be_ref,
               y_ref, z_ref):
    m = st_ref[0:1, :] / n_nodes
    var = st_ref[1:2, :] / n_nodes - m * m
    scale = lax.rsqrt(var + _EPS) * g_ref[0:1, :]
    h = jnp.maximum((h_ref[...] - m) * scale + be_ref[0:1, :], 0.0)
    y_ref[...] = jnp.dot(h, wl_ref[...], preferred_element_type=jnp.float32)
    z_ref[...] = (jnp.dot(h, wr_ref[...], preferred_element_type=jnp.float32)
                  + b_ref[0:1, :])


def _prologue(h, wlT, wrT, b, stats=None, g=None, be=None):
    n, d = h.shape
    wo = wlT.shape[1]
    grid = (n // _ROWS,)
    row_spec = pl.BlockSpec((_ROWS, d), lambda i: (i, 0))
    w_spec = pl.BlockSpec((d, wo), lambda i: (0, 0))
    vec_spec = pl.BlockSpec((1, wo), lambda i: (0, 0))
    out_spec = pl.BlockSpec((_ROWS, wo), lambda i: (i, 0))
    out_shape = [jax.ShapeDtypeStruct((n, wo), jnp.float32)] * 2
    if stats is None:
        body = functools.partial(_p_plain_body, n)
        in_specs = [row_spec, w_spec, w_spec, vec_spec]
        args = (h, wlT, wrT, b)
    else:
        body = functools.partial(_p_bn_body, n)
        dvec = pl.BlockSpec((1, d), lambda i: (0, 0))
        in_specs = [row_spec, w_spec, w_spec, vec_spec,
                    pl.BlockSpec((8, d), lambda i: (0, 0)), dvec, dvec]
        args = (h, wlT, wrT, b, stats, g, be)
    return pl.pallas_call(body, grid=grid, in_specs=in_specs,
                          out_specs=[out_spec, out_spec],
                          out_shape=out_shape)(*args)


def _e_stats_body(a0_ref, a1_ref, z_ref, d0_ref, d1_ref, h_ref, st_ref):
    deg = jnp.maximum(d0_ref[:, 0:1] + d1_ref[:, 0:1], 1.0)
    h = (a0_ref[...] + a1_ref[...]) / deg + z_ref[...]
    h_ref[...] = h

    @pl.when(pl.program_id(0) == 0)
    def _():
        st_ref[...] = jnp.zeros_like(st_ref)

    st_ref[0:1, :] += jnp.sum(h, axis=0, keepdims=True)
    st_ref[1:2, :] += jnp.sum(h * h, axis=0, keepdims=True)


def _e_plain_body(a0_ref, a1_ref, z_ref, d0_ref, d1_ref, h_ref):
    deg = jnp.maximum(d0_ref[:, 0:1] + d1_ref[:, 0:1], 1.0)
    h_ref[...] = (a0_ref[...] + a1_ref[...]) / deg + z_ref[...]


def _epilogue(accs, z, degs, with_stats):
    n, wo = z.shape
    nb = n // _ROWS
    grid = (nb,)
    a0 = pl.BlockSpec((_ROWS, wo), lambda i: (i, 0))
    a1 = pl.BlockSpec((_ROWS, wo), lambda i: (i + nb, 0))
    d0 = pl.BlockSpec((_ROWS, 128), lambda i: (i, 0))
    d1 = pl.BlockSpec((_ROWS, 128), lambda i: (i + nb, 0))
    zs = pl.BlockSpec((_ROWS, wo), lambda i: (i, 0))
    out_shape = [jax.ShapeDtypeStruct((n, wo), jnp.float32)]
    out_specs = [pl.BlockSpec((_ROWS, wo), lambda i: (i, 0))]
    body = _e_stats_body if with_stats else _e_plain_body
    if with_stats:
        out_shape.append(jax.ShapeDtypeStruct((8, 128), jnp.float32))
        out_specs.append(pl.BlockSpec((8, 128), lambda i: (0, 0)))
    return pl.pallas_call(body, grid=grid,
                          in_specs=[a0, a1, zs, d0, d1],
                          out_specs=out_specs,
                          out_shape=out_shape)(accs, accs, z, degs, degs)


def _make_sc_agg(n_nodes, width, e_total, ones_mode):
    nw = 32
    epw = e_total // nw
    nch = epw // _CHUNK
    rpt = (n_nodes // 16) // 8 * 8
    tail = n_nodes - 16 * rpt
    mesh = plsc.VectorSubcoreMesh(core_axis_name="c", subcore_axis_name="s")

    out_type = [jax.ShapeDtypeStruct((2 * n_nodes, width), jnp.float32)]
    scratch = [
        pltpu.VMEM((_CHUNK,), jnp.int32),
        pltpu.VMEM((_CHUNK,), jnp.int32),
        pltpu.VMEM((_CHUNK,), jnp.int32),
        pltpu.VMEM((_CHUNK,), jnp.int32),
        pltpu.VMEM((_CHUNK, width), jnp.float32),
        pltpu.VMEM((_CHUNK, width), jnp.float32),
        pltpu.VMEM_SHARED((n_nodes, width), jnp.float32),
        pltpu.SemaphoreType.DMA,
        pltpu.SemaphoreType.DMA,
    ]

    def body(*refs):
        if ones_mode:
            (src_hbm, dst_hbm, zf_hbm, acc_out,
             s0, s1, d0, d1, r0v, r1v, acc_sh, sm0, sm1) = refs
        else:
            (y_hbm, src_hbm, dst_hbm, zf_hbm, acc_out,
             s0, s1, d0, d1, r0v, r1v, acc_sh, sm0, sm1) = refs
        srcb, dstb, rowb, semb = (s0, s1), (d0, d1), (r0v, r1v), (sm0, sm1)
        c = lax.axis_index("c")
        s = lax.axis_index("s")
        w = c * 16 + s
        r0 = s * rpt
        t0 = 16 * rpt
        pltpu.sync_copy(zf_hbm.at[pl.ds(r0, rpt)], acc_sh.at[pl.ds(r0, rpt)])
        if tail:
            @pl.when(s == 15)
            def _():
                pltpu.sync_copy(zf_hbm.at[pl.ds(t0, tail)],
                                acc_sh.at[pl.ds(t0, tail)])
        if ones_mode:
            ov = jnp.full((16,), 1.0, jnp.float32)

            def fill_o(j, carry):
                for k in range(width // 16):
                    r0v[j, pl.ds(k * 16, 16)] = ov
                return carry

            lax.fori_loop(0, _CHUNK, fill_o, 0)
        plsc.subcore_barrier()

        base0 = w * epw

        if ones_mode:
            pltpu.sync_copy(dst_hbm.at[pl.ds(base0, _CHUNK)], d0)

            def step1(j2, carry):
                for b in range(2):
                    j = 2 * j2 + b

                    @pl.when(j + 1 < nch)
                    def _():
                        nb_ = base0 + (j + 1) * _CHUNK
                        pltpu.sync_copy(dst_hbm.at[pl.ds(nb_, _CHUNK)],
                                        dstb[1 - b])
                    pltpu.sync_copy(r0v, acc_sh.at[dstb[b]], add=True)
                return carry

            lax.fori_loop(0, nch // 2, step1, 0)
            if nch % 2:
                pltpu.sync_copy(r0v, acc_sh.at[dstb[(nch - 1) % 2]], add=True)
        else:
            def start(j, b):
                base = base0 + j * _CHUNK
                pltpu.sync_copy(src_hbm.at[pl.ds(base, _CHUNK)], srcb[b])
                pltpu.sync_copy(dst_hbm.at[pl.ds(base, _CHUNK)], dstb[b])
                pltpu.async_copy(y_hbm.at[srcb[b]], rowb[b], semb[b])

            def finish(b):
                pltpu.make_async_copy(y_hbm.at[srcb[b]], rowb[b],
                                      semb[b]).wait()
                pltpu.sync_copy(rowb[b], acc_sh.at[dstb[b]], add=True)

            start(0, 0)

            def step2(j2, carry):
                for b in range(2):
                    j = 2 * j2 + b

                    @pl.when(j + 1 < nch)
                    def _():
                        start(j + 1, 1 - b)
                    finish(b)
                return carry

            lax.fori_loop(0, nch // 2, step2, 0)
            if nch % 2:
                finish((nch - 1) % 2)
        plsc.subcore_barrier()
        o0 = c * n_nodes + r0
        pltpu.sync_copy(acc_sh.at[pl.ds(r0, rpt)], acc_out.at[pl.ds(o0, rpt)])
        if tail:
            @pl.when(s == 15)
            def _():
                ot = c * n_nodes + t0
                pltpu.sync_copy(acc_sh.at[pl.ds(t0, tail)],
                                acc_out.at[pl.ds(ot, tail)])

    return pl.kernel(body, out_type=tuple(out_type), mesh=mesh,
                     scratch_types=tuple(scratch))


def kernel(x, edge_index, Wl1, Wr1, b1, g1, be1, Wl2, Wr2, b2, g2, be2,
           Wl3, Wr3, b3):
    n, d = x.shape
    e = edge_index.shape[1]
    assert e % (32 * _CHUNK) == 0 and n % 16 == 0 and n % _ROWS == 0

    src = edge_index[0]
    dst = edge_index[1]
    zf = jnp.zeros((n, d), jnp.float32)

    agg_ones = _make_sc_agg(n, d, e, True)
    agg = _make_sc_agg(n, d, e, False)

    (degs,) = agg_ones(src, dst, zf)

    y1, zz1 = _prologue(x, Wl1.T, Wr1.T, b1.reshape(1, -1))
    (acc1,) = agg(y1, src, dst, zf)
    h1, st1 = _epilogue(acc1, zz1, degs, True)

    y2, zz2 = _prologue(h1, Wl2.T, Wr2.T, b2.reshape(1, -1),
                        st1, g1.reshape(1, -1), be1.reshape(1, -1))
    (acc2,) = agg(y2, src, dst, zf)
    h2, st2 = _epilogue(acc2, zz2, degs, True)

    w3l = jnp.pad(Wl3.T, ((0, 0), (0, d - 1)))
    w3r = jnp.pad(Wr3.T, ((0, 0), (0, d - 1)))
    b3w = jnp.pad(b3.reshape(1, 1), ((0, 0), (0, d - 1)))
    y3, zz3 = _prologue(h2, w3l, w3r, b3w,
                        st2, g2.reshape(1, -1), be2.reshape(1, -1))
    (acc3,) = agg(y3, src, dst, zf)
    (out_w,) = _epilogue(acc3, zz3, degs, False)
    return out_w[:, 0:1]


# speedup vs baseline: 9.5246x; 1.3731x over previous
import functools

import jax
import jax.numpy as jnp
from jax import lax
from jax.experimental import pallas as pl
from jax.experimental.pallas import tpu as pltpu
from jax.experimental.pallas import tpu_sc as plsc

_ROWS = 1000
_CHUNK = 80
_EPS = 1e-5


def _p_plain_body(n_nodes, h_ref, wl_ref, wr_ref, b_ref, y_ref, z_ref):
    h = h_ref[...]
    y_ref[...] = jnp.dot(h, wl_ref[...], preferred_element_type=jnp.float32)
    z_ref[...] = (jnp.dot(h, wr_ref[...], preferred_element_type=jnp.float32)
                  + b_ref[0:1, :])


def _p_bn_body(n_nodes, h_ref, wl_ref, wr_ref, b_ref, st_ref, g_ref, be_ref,
               y_ref, z_ref):
    m = st_ref[0:1, :] / n_nodes
    var = st_ref[1:2, :] / n_nodes - m * m
    scale = lax.rsqrt(var + _EPS) * g_ref[0:1, :]
    h = jnp.maximum((h_ref[...] - m) * scale + be_ref[0:1, :], 0.0)
    y_ref[...] = jnp.dot(h, wl_ref[...], preferred_element_type=jnp.float32)
    z_ref[...] = (jnp.dot(h, wr_ref[...], preferred_element_type=jnp.float32)
                  + b_ref[0:1, :])


def _prologue(h, wlT, wrT, b, stats=None, g=None, be=None):
    n, d = h.shape
    wo = wlT.shape[1]
    grid = (n // _ROWS,)
    row_spec = pl.BlockSpec((_ROWS, d), lambda i: (i, 0))
    w_spec = pl.BlockSpec((d, wo), lambda i: (0, 0))
    vec_spec = pl.BlockSpec((1, wo), lambda i: (0, 0))
    out_spec = pl.BlockSpec((_ROWS, wo), lambda i: (i, 0))
    out_shape = [jax.ShapeDtypeStruct((n, wo), jnp.float32)] * 2
    if stats is None:
        body = functools.partial(_p_plain_body, n)
        in_specs = [row_spec, w_spec, w_spec, vec_spec]
        args = (h, wlT, wrT, b)
    else:
        body = functools.partial(_p_bn_body, n)
        dvec = pl.BlockSpec((1, d), lambda i: (0, 0))
        in_specs = [row_spec, w_spec, w_spec, vec_spec,
                    pl.BlockSpec((8, d), lambda i: (0, 0)), dvec, dvec]
        args = (h, wlT, wrT, b, stats, g, be)
    return pl.pallas_call(body, grid=grid, in_specs=in_specs,
                          out_specs=[out_spec, out_spec],
                          out_shape=out_shape)(*args)


def _e_stats_body(a0_ref, a1_ref, z_ref, d0_ref, d1_ref, h_ref, st_ref):
    deg = jnp.maximum(d0_ref[...] + d1_ref[...], 1.0)
    h = (a0_ref[...] + a1_ref[...]) / deg + z_ref[...]
    h_ref[...] = h

    @pl.when(pl.program_id(0) == 0)
    def _():
        st_ref[...] = jnp.zeros_like(st_ref)

    st_ref[0:1, :] += jnp.sum(h, axis=0, keepdims=True)
    st_ref[1:2, :] += jnp.sum(h * h, axis=0, keepdims=True)


def _e_plain_body(a0_ref, a1_ref, z_ref, d0_ref, d1_ref, h_ref):
    deg = jnp.maximum(d0_ref[...] + d1_ref[...], 1.0)
    h_ref[...] = (a0_ref[...] + a1_ref[...]) / deg + z_ref[...]


def _epilogue(accs, z, degs, with_stats):
    n, wo = z.shape
    nb = n // _ROWS
    grid = (nb,)
    a0 = pl.BlockSpec((_ROWS, wo), lambda i: (i, 0))
    a1 = pl.BlockSpec((_ROWS, wo), lambda i: (i + nb, 0))
    d0 = pl.BlockSpec((_ROWS, 1), lambda i: (i, 0))
    d1 = pl.BlockSpec((_ROWS, 1), lambda i: (i + nb, 0))
    zs = pl.BlockSpec((_ROWS, wo), lambda i: (i, 0))
    out_shape = [jax.ShapeDtypeStruct((n, wo), jnp.float32)]
    out_specs = [pl.BlockSpec((_ROWS, wo), lambda i: (i, 0))]
    body = _e_stats_body if with_stats else _e_plain_body
    if with_stats:
        out_shape.append(jax.ShapeDtypeStruct((8, 128), jnp.float32))
        out_specs.append(pl.BlockSpec((8, 128), lambda i: (0, 0)))
    return pl.pallas_call(body, grid=grid,
                          in_specs=[a0, a1, zs, d0, d1],
                          out_specs=out_specs,
                          out_shape=out_shape)(accs, accs, z, degs, degs)


def _make_sc_agg(n_nodes, width, e_total, with_deg):
    nw = 32
    epw = e_total // nw
    nch = epw // _CHUNK
    rpt = (n_nodes // 16) // 8 * 8
    tail = n_nodes - 16 * rpt
    mesh = plsc.VectorSubcoreMesh(core_axis_name="c", subcore_axis_name="s")

    out_type = [jax.ShapeDtypeStruct((2 * n_nodes, width), jnp.float32)]
    scratch = [
        pltpu.VMEM((_CHUNK,), jnp.int32),
        pltpu.VMEM((_CHUNK,), jnp.int32),
        pltpu.VMEM((_CHUNK,), jnp.int32),
        pltpu.VMEM((_CHUNK,), jnp.int32),
        pltpu.VMEM((_CHUNK, width), jnp.float32),
        pltpu.VMEM((_CHUNK, width), jnp.float32),
        pltpu.VMEM_SHARED((n_nodes, width), jnp.float32),
        pltpu.SemaphoreType.DMA,
        pltpu.SemaphoreType.DMA,
        pltpu.SemaphoreType.DMA,
        pltpu.SemaphoreType.DMA,
    ]
    if with_deg:
        out_type.append(jax.ShapeDtypeStruct((2 * n_nodes,), jnp.float32))
        scratch += [
            pltpu.VMEM((_CHUNK,), jnp.float32),
            pltpu.VMEM((rpt,), jnp.float32),
            pltpu.VMEM_SHARED((n_nodes,), jnp.float32),
        ]

    def body(*refs):
        if with_deg:
            (y_hbm, src_hbm, dst_hbm, zf_hbm, acc_out, deg_out,
             s0, s1, d0, d1, r0v, r1v, acc_sh, gm0, gm1, im0, im1,
             ones_v, zb_v, deg_sh) = refs
        else:
            (y_hbm, src_hbm, dst_hbm, zf_hbm, acc_out,
             s0, s1, d0, d1, r0v, r1v, acc_sh, gm0, gm1, im0, im1) = refs
        srcb, dstb, rowb = (s0, s1), (d0, d1), (r0v, r1v)
        gsem, isem = (gm0, gm1), (im0, im1)
        c = lax.axis_index("c")
        s = lax.axis_index("s")
        w = c * 16 + s
        r0 = s * rpt
        t0 = 16 * rpt
        pltpu.sync_copy(zf_hbm.at[pl.ds(r0, rpt)], acc_sh.at[pl.ds(r0, rpt)])
        if with_deg:
            zv = jnp.zeros((16,), jnp.float32)
            ov = jnp.full((16,), 1.0, jnp.float32)
            for k in range(_CHUNK // 16):
                ones_v[pl.ds(k * 16, 16)] = ov

            def fill_z(j, carry):
                zb_v[pl.ds(j * 16, 16)] = zv
                return carry

            lax.fori_loop(0, rpt // 16, fill_z, 0)
            pltpu.sync_copy(zb_v, deg_sh.at[pl.ds(r0, rpt)])
        if tail:
            @pl.when(s == 15)
            def _():
                pltpu.sync_copy(zf_hbm.at[pl.ds(t0, tail)],
                                acc_sh.at[pl.ds(t0, tail)])
                if with_deg:
                    pltpu.sync_copy(zb_v.at[pl.ds(0, tail)],
                                    deg_sh.at[pl.ds(t0, tail)])
        plsc.subcore_barrier()

        base0 = w * epw

        def start_idx(j, b):
            base = base0 + j * _CHUNK
            pltpu.async_copy(src_hbm.at[pl.ds(base, _CHUNK)], srcb[b], isem[b])
            pltpu.async_copy(dst_hbm.at[pl.ds(base, _CHUNK)], dstb[b], isem[b])

        def wait_idx(j, b):
            base = base0 + j * _CHUNK
            pltpu.make_async_copy(src_hbm.at[pl.ds(base, _CHUNK)], srcb[b],
                                  isem[b]).wait()
            pltpu.make_async_copy(dst_hbm.at[pl.ds(base, _CHUNK)], dstb[b],
                                  isem[b]).wait()

        def start_gather(b):
            pltpu.async_copy(y_hbm.at[srcb[b]], rowb[b], gsem[b])

        def finish(b):
            pltpu.make_async_copy(y_hbm.at[srcb[b]], rowb[b], gsem[b]).wait()
            pltpu.sync_copy(rowb[b], acc_sh.at[dstb[b]], add=True)
            if with_deg:
                pltpu.sync_copy(ones_v, deg_sh.at[dstb[b]], add=True)

        start_idx(0, 0)
        start_idx(1, 1)
        wait_idx(0, 0)
        start_gather(0)

        def step2(j2, carry):
            for b in range(2):
                j = 2 * j2 + b

                @pl.when(j + 1 < nch)
                def _():
                    wait_idx(j + 1, 1 - b)
                    start_gather(1 - b)
                finish(b)

                @pl.when(j + 2 < nch)
                def _():
                    start_idx(j + 2, b)
            return carry

        lax.fori_loop(0, nch // 2, step2, 0)
        if nch % 2:
            finish((nch - 1) % 2)
        plsc.subcore_barrier()
        o0 = c * n_nodes + r0
        pltpu.sync_copy(acc_sh.at[pl.ds(r0, rpt)], acc_out.at[pl.ds(o0, rpt)])
        if with_deg:
            pltpu.sync_copy(deg_sh.at[pl.ds(r0, rpt)], zb_v)
            pltpu.sync_copy(zb_v, deg_out.at[pl.ds(c * n_nodes + r0, rpt)])
        if tail:
            @pl.when(s == 15)
            def _():
                ot = c * n_nodes + t0
                pltpu.sync_copy(acc_sh.at[pl.ds(t0, tail)],
                                acc_out.at[pl.ds(ot, tail)])
                if with_deg:
                    pltpu.sync_copy(deg_sh.at[pl.ds(t0, tail)],
                                    zb_v.at[pl.ds(0, tail)])
                    pltpu.sync_copy(zb_v.at[pl.ds(0, tail)],
                                    deg_out.at[pl.ds(ot, tail)])

    return pl.kernel(body, out_type=tuple(out_type), mesh=mesh,
                     scratch_types=tuple(scratch))


def kernel(x, edge_index, Wl1, Wr1, b1, g1, be1, Wl2, Wr2, b2, g2, be2,
           Wl3, Wr3, b3):
    n, d = x.shape
    e = edge_index.shape[1]
    assert e % (32 * _CHUNK) == 0 and n % 16 == 0 and n % _ROWS == 0

    src = edge_index[0]
    dst = edge_index[1]
    zf = jnp.zeros((n, d), jnp.float32)

    agg_deg = _make_sc_agg(n, d, e, True)
    agg = _make_sc_agg(n, d, e, False)

    y1, zz1 = _prologue(x, Wl1.T, Wr1.T, b1.reshape(1, -1))
    acc1, deg_flat = agg_deg(y1, src, dst, zf)
    degs = deg_flat.reshape(2 * n, 1)
    h1, st1 = _epilogue(acc1, zz1, degs, True)

    y2, zz2 = _prologue(h1, Wl2.T, Wr2.T, b2.reshape(1, -1),
                        st1, g1.reshape(1, -1), be1.reshape(1, -1))
    (acc2,) = agg(y2, src, dst, zf)
    h2, st2 = _epilogue(acc2, zz2, degs, True)

    w3l = jnp.pad(Wl3.T, ((0, 0), (0, d - 1)))
    w3r = jnp.pad(Wr3.T, ((0, 0), (0, d - 1)))
    b3w = jnp.pad(b3.reshape(1, 1), ((0, 0), (0, d - 1)))
    y3, zz3 = _prologue(h2, w3l, w3r, b3w,
                        st2, g2.reshape(1, -1), be2.reshape(1, -1))
    (acc3,) = agg(y3, src, dst, zf)
    (out_w,) = _epilogue(acc3, zz3, degs, False)
    return out_w[:, 0:1]


# speedup vs baseline: 11.2475x; 1.1809x over previous
import functools

import jax
import jax.numpy as jnp
from jax import lax
from jax.experimental import pallas as pl
from jax.experimental.pallas import tpu as pltpu
from jax.experimental.pallas import tpu_sc as plsc

_ROWS = 1000
_CHUNK = 80
_EPS = 1e-5


def _p_plain_body(n_nodes, h_ref, wl_ref, wr_ref, b_ref, y_ref, z_ref):
    h = h_ref[...]
    y_ref[...] = jnp.dot(h, wl_ref[...], preferred_element_type=jnp.float32)
    z_ref[...] = (jnp.dot(h, wr_ref[...], preferred_element_type=jnp.float32)
                  + b_ref[0:1, :])


def _p_bn_body(n_nodes, h_ref, wl_ref, wr_ref, b_ref, st_ref, g_ref, be_ref,
               y_ref, z_ref):
    m = st_ref[0:1, :] / n_nodes
    var = st_ref[1:2, :] / n_nodes - m * m
    scale = lax.rsqrt(var + _EPS) * g_ref[0:1, :]
    h = jnp.maximum((h_ref[...] - m) * scale + be_ref[0:1, :], 0.0)
    y_ref[...] = jnp.dot(h, wl_ref[...], preferred_element_type=jnp.float32)
    z_ref[...] = (jnp.dot(h, wr_ref[...], preferred_element_type=jnp.float32)
                  + b_ref[0:1, :])


def _prologue(h, wlT, wrT, b, stats=None, g=None, be=None):
    n, d = h.shape
    wo = wlT.shape[1]
    grid = (n // _ROWS,)
    row_spec = pl.BlockSpec((_ROWS, d), lambda i: (i, 0))
    w_spec = pl.BlockSpec((d, wo), lambda i: (0, 0))
    vec_spec = pl.BlockSpec((1, wo), lambda i: (0, 0))
    out_spec = pl.BlockSpec((_ROWS, wo), lambda i: (i, 0))
    out_shape = [jax.ShapeDtypeStruct((n, wo), jnp.float32)] * 2
    if stats is None:
        body = functools.partial(_p_plain_body, n)
        in_specs = [row_spec, w_spec, w_spec, vec_spec]
        args = (h, wlT, wrT, b)
    else:
        body = functools.partial(_p_bn_body, n)
        dvec = pl.BlockSpec((1, d), lambda i: (0, 0))
        in_specs = [row_spec, w_spec, w_spec, vec_spec,
                    pl.BlockSpec((8, d), lambda i: (0, 0)), dvec, dvec]
        args = (h, wlT, wrT, b, stats, g, be)
    return pl.pallas_call(body, grid=grid, in_specs=in_specs,
                          out_specs=[out_spec, out_spec],
                          out_shape=out_shape)(*args)


def _e_stats_body(a0_ref, a1_ref, z_ref, d0_ref, d1_ref, h_ref, st_ref):
    deg = jnp.maximum(d0_ref[...] + d1_ref[...], 1.0)
    h = (a0_ref[...] + a1_ref[...]) / deg + z_ref[...]
    h_ref[...] = h

    @pl.when(pl.program_id(0) == 0)
    def _():
        st_ref[...] = jnp.zeros_like(st_ref)

    st_ref[0:1, :] += jnp.sum(h, axis=0, keepdims=True)
    st_ref[1:2, :] += jnp.sum(h * h, axis=0, keepdims=True)


def _e_plain_body(a0_ref, a1_ref, z_ref, d0_ref, d1_ref, h_ref):
    deg = jnp.maximum(d0_ref[...] + d1_ref[...], 1.0)
    h_ref[...] = (a0_ref[...] + a1_ref[...]) / deg + z_ref[...]


def _epilogue(accs, z, degs, with_stats):
    n, wo = z.shape
    nb = n // _ROWS
    grid = (nb,)
    a0 = pl.BlockSpec((_ROWS, wo), lambda i: (i, 0))
    a1 = pl.BlockSpec((_ROWS, wo), lambda i: (i + nb, 0))
    d0 = pl.BlockSpec((_ROWS, 1), lambda i: (i, 0))
    d1 = pl.BlockSpec((_ROWS, 1), lambda i: (i + nb, 0))
    zs = pl.BlockSpec((_ROWS, wo), lambda i: (i, 0))
    out_shape = [jax.ShapeDtypeStruct((n, wo), jnp.float32)]
    out_specs = [pl.BlockSpec((_ROWS, wo), lambda i: (i, 0))]
    body = _e_stats_body if with_stats else _e_plain_body
    if with_stats:
        out_shape.append(jax.ShapeDtypeStruct((8, 128), jnp.float32))
        out_specs.append(pl.BlockSpec((8, 128), lambda i: (0, 0)))
    return pl.pallas_call(body, grid=grid,
                          in_specs=[a0, a1, zs, d0, d1],
                          out_specs=out_specs,
                          out_shape=out_shape)(accs, accs, z, degs, degs)


def _make_sc_agg(n_nodes, width, e_total, with_deg):
    nw = 32
    epw = e_total // nw
    nch = epw // _CHUNK
    rpt = (n_nodes // 16) // 8 * 8
    tail = n_nodes - 16 * rpt
    mesh = plsc.VectorSubcoreMesh(core_axis_name="c", subcore_axis_name="s")

    nbuf = 3
    out_type = [jax.ShapeDtypeStruct((2 * n_nodes, width), jnp.float32)]
    scratch = (
        [pltpu.VMEM((_CHUNK,), jnp.int32) for _ in range(nbuf)]
        + [pltpu.VMEM((_CHUNK,), jnp.int32) for _ in range(nbuf)]
        + [pltpu.VMEM((_CHUNK, width), jnp.float32) for _ in range(nbuf)]
        + [pltpu.VMEM_SHARED((n_nodes, width), jnp.float32)]
        + [pltpu.SemaphoreType.DMA for _ in range(3 * nbuf)]
    )
    if with_deg:
        out_type.append(jax.ShapeDtypeStruct((2 * n_nodes,), jnp.float32))
        scratch += [
            pltpu.VMEM((_CHUNK,), jnp.float32),
            pltpu.VMEM((rpt,), jnp.float32),
            pltpu.VMEM_SHARED((n_nodes,), jnp.float32),
        ]

    def body(*refs):
        if with_deg:
            (y_hbm, src_hbm, dst_hbm, zf_hbm, acc_out, deg_out) = refs[:6]
            (ones_v, zb_v, deg_sh) = refs[-3:]
            rest = refs[6:-3]
        else:
            (y_hbm, src_hbm, dst_hbm, zf_hbm, acc_out) = refs[:5]
            rest = refs[5:]
        srcb = rest[0:nbuf]
        dstb = rest[nbuf:2 * nbuf]
        rowb = rest[2 * nbuf:3 * nbuf]
        acc_sh = rest[3 * nbuf]
        gsem = rest[3 * nbuf + 1:3 * nbuf + 1 + nbuf]
        isem = rest[3 * nbuf + 1 + nbuf:3 * nbuf + 1 + 2 * nbuf]
        ssem = rest[3 * nbuf + 1 + 2 * nbuf:3 * nbuf + 1 + 3 * nbuf]
        c = lax.axis_index("c")
        s = lax.axis_index("s")
        w = c * 16 + s
        r0 = s * rpt
        t0 = 16 * rpt
        pltpu.sync_copy(zf_hbm.at[pl.ds(r0, rpt)], acc_sh.at[pl.ds(r0, rpt)])
        if with_deg:
            zv = jnp.zeros((16,), jnp.float32)
            ov = jnp.full((16,), 1.0, jnp.float32)
            for k in range(_CHUNK // 16):
                ones_v[pl.ds(k * 16, 16)] = ov

            def fill_z(j, carry):
                zb_v[pl.ds(j * 16, 16)] = zv
                return carry

            lax.fori_loop(0, rpt // 16, fill_z, 0)
            pltpu.sync_copy(zb_v, deg_sh.at[pl.ds(r0, rpt)])
        if tail:
            @pl.when(s == 15)
            def _():
                pltpu.sync_copy(zf_hbm.at[pl.ds(t0, tail)],
                                acc_sh.at[pl.ds(t0, tail)])
                if with_deg:
                    pltpu.sync_copy(zb_v.at[pl.ds(0, tail)],
                                    deg_sh.at[pl.ds(t0, tail)])
        plsc.subcore_barrier()

        base0 = w * epw

        def start_idx(j, b):
            base = base0 + j * _CHUNK
            pltpu.async_copy(src_hbm.at[pl.ds(base, _CHUNK)], srcb[b], isem[b])
            pltpu.async_copy(dst_hbm.at[pl.ds(base, _CHUNK)], dstb[b], isem[b])

        def wait_idx(j, b):
            base = base0 + j * _CHUNK
            pltpu.make_async_copy(src_hbm.at[pl.ds(base, _CHUNK)], srcb[b],
                                  isem[b]).wait()
            pltpu.make_async_copy(dst_hbm.at[pl.ds(base, _CHUNK)], dstb[b],
                                  isem[b]).wait()

        def start_gather(b):
            pltpu.async_copy(y_hbm.at[srcb[b]], rowb[b], gsem[b])

        def wait_gather(b):
            pltpu.make_async_copy(y_hbm.at[srcb[b]], rowb[b], gsem[b]).wait()

        def start_scatter(b):
            pltpu.async_copy(rowb[b], acc_sh.at[dstb[b]], ssem[b], add=True)
            if with_deg:
                pltpu.async_copy(ones_v, deg_sh.at[dstb[b]], ssem[b], add=True)

        def wait_scatter(b):
            pltpu.make_async_copy(rowb[b], acc_sh.at[dstb[b]], ssem[b]).wait()
            if with_deg:
                pltpu.make_async_copy(ones_v, deg_sh.at[dstb[b]],
                                      ssem[b]).wait()

        def stage(j, b, p, q, guard):
            if guard:
                @pl.when(j >= 1)
                def _():
                    wait_scatter(q)
            else:
                if j >= 1:
                    wait_scatter(q)

            @pl.when(j + 2 < nch)
            def _():
                start_idx(j + 2, q)

            @pl.when(j + 1 < nch)
            def _():
                wait_idx(j + 1, p)
                start_gather(p)
            wait_gather(b)
            start_scatter(b)

        start_idx(0, 0)
        start_idx(1, 1)
        wait_idx(0, 0)
        start_gather(0)

        def step3(j3, carry):
            for b in range(nbuf):
                j = nbuf * j3 + b
                stage(j, b, (b + 1) % nbuf, (b + 2) % nbuf, True)
            return carry

        nfull = nch // nbuf
        lax.fori_loop(0, nfull, step3, 0)
        for j in range(nfull * nbuf, nch):
            b = j % nbuf
            stage(j, b, (j + 1) % nbuf, (j + 2) % nbuf, False)
        wait_scatter((nch - 1) % nbuf)
        plsc.subcore_barrier()
        o0 = c * n_nodes + r0
        pltpu.sync_copy(acc_sh.at[pl.ds(r0, rpt)], acc_out.at[pl.ds(o0, rpt)])
        if with_deg:
            pltpu.sync_copy(deg_sh.at[pl.ds(r0, rpt)], zb_v)
            pltpu.sync_copy(zb_v, deg_out.at[pl.ds(c * n_nodes + r0, rpt)])
        if tail:
            @pl.when(s == 15)
            def _():
                ot = c * n_nodes + t0
                pltpu.sync_copy(acc_sh.at[pl.ds(t0, tail)],
                                acc_out.at[pl.ds(ot, tail)])
                if with_deg:
                    pltpu.sync_copy(deg_sh.at[pl.ds(t0, tail)],
                                    zb_v.at[pl.ds(0, tail)])
                    pltpu.sync_copy(zb_v.at[pl.ds(0, tail)],
                                    deg_out.at[pl.ds(ot, tail)])

    return pl.kernel(body, out_type=tuple(out_type), mesh=mesh,
                     scratch_types=tuple(scratch))


def kernel(x, edge_index, Wl1, Wr1, b1, g1, be1, Wl2, Wr2, b2, g2, be2,
           Wl3, Wr3, b3):
    n, d = x.shape
    e = edge_index.shape[1]
    assert e % (32 * _CHUNK) == 0 and n % 16 == 0 and n % _ROWS == 0

    src = edge_index[0]
    dst = edge_index[1]
    zf = jnp.zeros((n, d), jnp.float32)

    agg_deg = _make_sc_agg(n, d, e, True)
    agg = _make_sc_agg(n, d, e, False)

    y1, zz1 = _prologue(x, Wl1.T, Wr1.T, b1.reshape(1, -1))
    acc1, deg_flat = agg_deg(y1, src, dst, zf)
    degs = deg_flat.reshape(2 * n, 1)
    h1, st1 = _epilogue(acc1, zz1, degs, True)

    y2, zz2 = _prologue(h1, Wl2.T, Wr2.T, b2.reshape(1, -1),
                        st1, g1.reshape(1, -1), be1.reshape(1, -1))
    (acc2,) = agg(y2, src, dst, zf)
    h2, st2 = _epilogue(acc2, zz2, degs, True)

    w3l = jnp.pad(Wl3.T, ((0, 0), (0, d - 1)))
    w3r = jnp.pad(Wr3.T, ((0, 0), (0, d - 1)))
    b3w = jnp.pad(b3.reshape(1, 1), ((0, 0), (0, d - 1)))
    y3, zz3 = _prologue(h2, w3l, w3r, b3w,
                        st2, g2.reshape(1, -1), be2.reshape(1, -1))
    (acc3,) = agg(y3, src, dst, zf)
    (out_w,) = _epilogue(acc3, zz3, degs, False)
    return out_w[:, 0:1]


# speedup vs baseline: 11.4468x; 1.0177x over previous
import functools

import jax
import jax.numpy as jnp
from jax import lax
from jax.experimental import pallas as pl
from jax.experimental.pallas import tpu as pltpu
from jax.experimental.pallas import tpu_sc as plsc

_ROWS = 1000
_CHUNK = 80
_EPS = 1e-5


def _p_plain_body(n_nodes, h_ref, wl_ref, wr_ref, b_ref, y_ref, z_ref):
    h = h_ref[...]
    y_ref[...] = jnp.dot(h, wl_ref[...], preferred_element_type=jnp.float32)
    z_ref[...] = (jnp.dot(h, wr_ref[...], preferred_element_type=jnp.float32)
                  + b_ref[0:1, :])


def _p_bn_body(n_nodes, h_ref, wl_ref, wr_ref, b_ref, st_ref, g_ref, be_ref,
               y_ref, z_ref):
    m = st_ref[0:1, :] / n_nodes
    var = st_ref[1:2, :] / n_nodes - m * m
    scale = lax.rsqrt(var + _EPS) * g_ref[0:1, :]
    h = jnp.maximum((h_ref[...] - m) * scale + be_ref[0:1, :], 0.0)
    y_ref[...] = jnp.dot(h, wl_ref[...], preferred_element_type=jnp.float32)
    z_ref[...] = (jnp.dot(h, wr_ref[...], preferred_element_type=jnp.float32)
                  + b_ref[0:1, :])


def _prologue(h, wlT, wrT, b, stats=None, g=None, be=None):
    n, d = h.shape
    wo = wlT.shape[1]
    grid = (n // _ROWS,)
    row_spec = pl.BlockSpec((_ROWS, d), lambda i: (i, 0))
    w_spec = pl.BlockSpec((d, wo), lambda i: (0, 0))
    vec_spec = pl.BlockSpec((1, wo), lambda i: (0, 0))
    out_spec = pl.BlockSpec((_ROWS, wo), lambda i: (i, 0))
    out_shape = [jax.ShapeDtypeStruct((n, wo), jnp.float32)] * 2
    if stats is None:
        body = functools.partial(_p_plain_body, n)
        in_specs = [row_spec, w_spec, w_spec, vec_spec]
        args = (h, wlT, wrT, b)
    else:
        body = functools.partial(_p_bn_body, n)
        dvec = pl.BlockSpec((1, d), lambda i: (0, 0))
        in_specs = [row_spec, w_spec, w_spec, vec_spec,
                    pl.BlockSpec((8, d), lambda i: (0, 0)), dvec, dvec]
        args = (h, wlT, wrT, b, stats, g, be)
    return pl.pallas_call(body, grid=grid, in_specs=in_specs,
                          out_specs=[out_spec, out_spec],
                          out_shape=out_shape)(*args)


def _e_stats_body(a0_ref, a1_ref, z_ref, d0_ref, d1_ref, h_ref, st_ref):
    deg = jnp.maximum(d0_ref[...] + d1_ref[...], 1.0)
    h = (a0_ref[...] + a1_ref[...]) / deg + z_ref[...]
    h_ref[...] = h

    @pl.when(pl.program_id(0) == 0)
    def _():
        st_ref[...] = jnp.zeros_like(st_ref)

    st_ref[0:1, :] += jnp.sum(h, axis=0, keepdims=True)
    st_ref[1:2, :] += jnp.sum(h * h, axis=0, keepdims=True)


def _e_plain_body(a0_ref, a1_ref, z_ref, d0_ref, d1_ref, h_ref):
    deg = jnp.maximum(d0_ref[...] + d1_ref[...], 1.0)
    h_ref[...] = (a0_ref[...] + a1_ref[...]) / deg + z_ref[...]


def _epilogue(accs, z, degs, with_stats):
    n, wo = z.shape
    nb = n // _ROWS
    grid = (nb,)
    a0 = pl.BlockSpec((_ROWS, wo), lambda i: (i, 0))
    a1 = pl.BlockSpec((_ROWS, wo), lambda i: (i + nb, 0))
    d0 = pl.BlockSpec((_ROWS, 1), lambda i: (i, 0))
    d1 = pl.BlockSpec((_ROWS, 1), lambda i: (i + nb, 0))
    zs = pl.BlockSpec((_ROWS, wo), lambda i: (i, 0))
    out_shape = [jax.ShapeDtypeStruct((n, wo), jnp.float32)]
    out_specs = [pl.BlockSpec((_ROWS, wo), lambda i: (i, 0))]
    body = _e_stats_body if with_stats else _e_plain_body
    if with_stats:
        out_shape.append(jax.ShapeDtypeStruct((8, 128), jnp.float32))
        out_specs.append(pl.BlockSpec((8, 128), lambda i: (0, 0)))
    return pl.pallas_call(body, grid=grid,
                          in_specs=[a0, a1, zs, d0, d1],
                          out_specs=out_specs,
                          out_shape=out_shape)(accs, accs, z, degs, degs)


def _m_body(n_nodes, a0_ref, a1_ref, z_ref, d0_ref, d1_ref, wl_ref, wr_ref,
            b_ref, g_ref, be_ref, y_ref, z2_ref, h_scr, st_scr):
    p = pl.program_id(0)
    i = pl.program_id(1)

    @pl.when(p == 0)
    def _():
        deg = jnp.maximum(d0_ref[...] + d1_ref[...], 1.0)
        h = (a0_ref[...] + a1_ref[...]) / deg + z_ref[...]
        h_scr[pl.ds(i * _ROWS, _ROWS), :] = h

        @pl.when(i == 0)
        def _():
            st_scr[...] = jnp.zeros_like(st_scr)

        st_scr[0:1, :] += jnp.sum(h, axis=0, keepdims=True)
        st_scr[1:2, :] += jnp.sum(h * h, axis=0, keepdims=True)

    @pl.when(p == 1)
    def _():
        m = st_scr[0:1, :] / n_nodes
        var = st_scr[1:2, :] / n_nodes - m * m
        scale = lax.rsqrt(var + _EPS) * g_ref[0:1, :]
        h = h_scr[pl.ds(i * _ROWS, _ROWS), :]
        h = jnp.maximum((h - m) * scale + be_ref[0:1, :], 0.0)
        y_ref[...] = jnp.dot(h, wl_ref[...], preferred_element_type=jnp.float32)
        z2_ref[...] = (jnp.dot(h, wr_ref[...],
                               preferred_element_type=jnp.float32)
                       + b_ref[0:1, :])


def _mid(accs, z, degs, wlT, wrT, b, g, be):
    n, dd = z.shape
    wo = wlT.shape[1]
    nb = n // _ROWS
    grid = (2, nb)

    def once(bs):
        return pl.BlockSpec(bs, lambda p, i: (0, 0))

    def p0(bs):
        return pl.BlockSpec(bs, lambda p, i: ((1 - p) * i, 0))

    def p1(bs):
        return pl.BlockSpec(bs, lambda p, i: (p * i, 0))

    a0 = pl.BlockSpec((_ROWS, dd), lambda p, i: ((1 - p) * i, 0))
    a1 = pl.BlockSpec((_ROWS, dd), lambda p, i: ((1 - p) * i + nb, 0))
    d0 = pl.BlockSpec((_ROWS, 1), lambda p, i: ((1 - p) * i, 0))
    d1 = pl.BlockSpec((_ROWS, 1), lambda p, i: ((1 - p) * i + nb, 0))
    out_shape = [jax.ShapeDtypeStruct((n, wo), jnp.float32)] * 2
    return pl.pallas_call(
        functools.partial(_m_body, n), grid=grid,
        in_specs=[a0, a1, p0((_ROWS, dd)), d0, d1,
                  once((dd, wo)), once((dd, wo)), once((1, wo)),
                  once((1, dd)), once((1, dd))],
        out_specs=[p1((_ROWS, wo)), p1((_ROWS, wo))],
        out_shape=out_shape,
        scratch_shapes=[pltpu.VMEM((n, dd), jnp.float32),
                        pltpu.VMEM((8, dd), jnp.float32)],
    )(accs, accs, z, degs, degs, wlT, wrT, b, g, be)


def _make_sc_agg(n_nodes, width, e_total, with_deg):
    nw = 32
    epw = e_total // nw
    nch = epw // _CHUNK
    rpt = (n_nodes // 16) // 8 * 8
    tail = n_nodes - 16 * rpt
    mesh = plsc.VectorSubcoreMesh(core_axis_name="c", subcore_axis_name="s")

    nbuf = 3
    out_type = [jax.ShapeDtypeStruct((2 * n_nodes, width), jnp.float32)]
    scratch = (
        [pltpu.VMEM((_CHUNK,), jnp.int32) for _ in range(nbuf)]
        + [pltpu.VMEM((_CHUNK,), jnp.int32) for _ in range(nbuf)]
        + [pltpu.VMEM((_CHUNK, width), jnp.float32) for _ in range(nbuf)]
        + [pltpu.VMEM_SHARED((n_nodes, width), jnp.float32)]
        + [pltpu.SemaphoreType.DMA for _ in range(3 * nbuf)]
    )
    if with_deg:
        out_type.append(jax.ShapeDtypeStruct((2 * n_nodes,), jnp.float32))
        scratch += [
            pltpu.VMEM((_CHUNK,), jnp.float32),
            pltpu.VMEM((rpt,), jnp.float32),
            pltpu.VMEM_SHARED((n_nodes,), jnp.float32),
        ]

    def body(*refs):
        if with_deg:
            (y_hbm, src_hbm, dst_hbm, zf_hbm, acc_out, deg_out) = refs[:6]
            (ones_v, zb_v, deg_sh) = refs[-3:]
            rest = refs[6:-3]
        else:
            (y_hbm, src_hbm, dst_hbm, zf_hbm, acc_out) = refs[:5]
            rest = refs[5:]
        srcb = rest[0:nbuf]
        dstb = rest[nbuf:2 * nbuf]
        rowb = rest[2 * nbuf:3 * nbuf]
        acc_sh = rest[3 * nbuf]
        gsem = rest[3 * nbuf + 1:3 * nbuf + 1 + nbuf]
        isem = rest[3 * nbuf + 1 + nbuf:3 * nbuf + 1 + 2 * nbuf]
        ssem = rest[3 * nbuf + 1 + 2 * nbuf:3 * nbuf + 1 + 3 * nbuf]
        c = lax.axis_index("c")
        s = lax.axis_index("s")
        w = c * 16 + s
        r0 = s * rpt
        t0 = 16 * rpt
        pltpu.sync_copy(zf_hbm.at[pl.ds(r0, rpt)], acc_sh.at[pl.ds(r0, rpt)])
        if with_deg:
            zv = jnp.zeros((16,), jnp.float32)
            ov = jnp.full((16,), 1.0, jnp.float32)
            for k in range(_CHUNK // 16):
                ones_v[pl.ds(k * 16, 16)] = ov

            def fill_z(j, carry):
                zb_v[pl.ds(j * 16, 16)] = zv
                return carry

            lax.fori_loop(0, rpt // 16, fill_z, 0)
            pltpu.sync_copy(zb_v, deg_sh.at[pl.ds(r0, rpt)])
        if tail:
            @pl.when(s == 15)
            def _():
                pltpu.sync_copy(zf_hbm.at[pl.ds(t0, tail)],
                                acc_sh.at[pl.ds(t0, tail)])
                if with_deg:
                    pltpu.sync_copy(zb_v.at[pl.ds(0, tail)],
                                    deg_sh.at[pl.ds(t0, tail)])
        plsc.subcore_barrier()

        base0 = w * epw

        def start_idx(j, b):
            base = base0 + j * _CHUNK
            pltpu.async_copy(src_hbm.at[pl.ds(base, _CHUNK)], srcb[b], isem[b])
            pltpu.async_copy(dst_hbm.at[pl.ds(base, _CHUNK)], dstb[b], isem[b])

        def wait_idx(j, b):
            base = base0 + j * _CHUNK
            pltpu.make_async_copy(src_hbm.at[pl.ds(base, _CHUNK)], srcb[b],
                                  isem[b]).wait()
            pltpu.make_async_copy(dst_hbm.at[pl.ds(base, _CHUNK)], dstb[b],
                                  isem[b]).wait()

        def start_gather(b):
            pltpu.async_copy(y_hbm.at[srcb[b]], rowb[b], gsem[b])

        def wait_gather(b):
            pltpu.make_async_copy(y_hbm.at[srcb[b]], rowb[b], gsem[b]).wait()

        def start_scatter(b):
            pltpu.async_copy(rowb[b], acc_sh.at[dstb[b]], ssem[b], add=True)
            if with_deg:
                pltpu.async_copy(ones_v, deg_sh.at[dstb[b]], ssem[b], add=True)

        def wait_scatter(b):
            pltpu.make_async_copy(rowb[b], acc_sh.at[dstb[b]], ssem[b]).wait()
            if with_deg:
                pltpu.make_async_copy(ones_v, deg_sh.at[dstb[b]],
                                      ssem[b]).wait()

        def stage(j, b, p, q, guard):
            if guard:
                @pl.when(j >= 1)
                def _():
                    wait_scatter(q)
            else:
                if j >= 1:
                    wait_scatter(q)

            @pl.when(j + 2 < nch)
            def _():
                start_idx(j + 2, q)

            @pl.when(j + 1 < nch)
            def _():
                wait_idx(j + 1, p)
                start_gather(p)
            wait_gather(b)
            start_scatter(b)

        start_idx(0, 0)
        start_idx(1, 1)
        wait_idx(0, 0)
        start_gather(0)

        def step3(j3, carry):
            for b in range(nbuf):
                j = nbuf * j3 + b
                stage(j, b, (b + 1) % nbuf, (b + 2) % nbuf, True)
            return carry

        nfull = nch // nbuf
        lax.fori_loop(0, nfull, step3, 0)
        for j in range(nfull * nbuf, nch):
            b = j % nbuf
            stage(j, b, (j + 1) % nbuf, (j + 2) % nbuf, False)
        wait_scatter((nch - 1) % nbuf)
        plsc.subcore_barrier()
        o0 = c * n_nodes + r0
        pltpu.sync_copy(acc_sh.at[pl.ds(r0, rpt)], acc_out.at[pl.ds(o0, rpt)])
        if with_deg:
            pltpu.sync_copy(deg_sh.at[pl.ds(r0, rpt)], zb_v)
            pltpu.sync_copy(zb_v, deg_out.at[pl.ds(c * n_nodes + r0, rpt)])
        if tail:
            @pl.when(s == 15)
            def _():
                ot = c * n_nodes + t0
                pltpu.sync_copy(acc_sh.at[pl.ds(t0, tail)],
                                acc_out.at[pl.ds(ot, tail)])
                if with_deg:
                    pltpu.sync_copy(deg_sh.at[pl.ds(t0, tail)],
                                    zb_v.at[pl.ds(0, tail)])
                    pltpu.sync_copy(zb_v.at[pl.ds(0, tail)],
                                    deg_out.at[pl.ds(ot, tail)])

    return pl.kernel(body, out_type=tuple(out_type), mesh=mesh,
                     scratch_types=tuple(scratch))


def kernel(x, edge_index, Wl1, Wr1, b1, g1, be1, Wl2, Wr2, b2, g2, be2,
           Wl3, Wr3, b3):
    n, d = x.shape
    e = edge_index.shape[1]
    assert e % (32 * _CHUNK) == 0 and n % 16 == 0 and n % _ROWS == 0

    src = edge_index[0]
    dst = edge_index[1]
    zf = jnp.zeros((n, d), jnp.float32)

    agg_deg = _make_sc_agg(n, d, e, True)
    agg = _make_sc_agg(n, d, e, False)

    y1, zz1 = _prologue(x, Wl1.T, Wr1.T, b1.reshape(1, -1))
    acc1, deg_flat = agg_deg(y1, src, dst, zf)
    degs = deg_flat.reshape(2 * n, 1)

    y2, zz2 = _mid(acc1, zz1, degs, Wl2.T, Wr2.T, b2.reshape(1, -1),
                   g1.reshape(1, -1), be1.reshape(1, -1))
    (acc2,) = agg(y2, src, dst, zf)

    w3l = jnp.pad(Wl3.T, ((0, 0), (0, d - 1)))
    w3r = jnp.pad(Wr3.T, ((0, 0), (0, d - 1)))
    b3w = jnp.pad(b3.reshape(1, 1), ((0, 0), (0, d - 1)))
    y3, zz3 = _mid(acc2, zz2, degs, w3l, w3r, b3w,
                   g2.reshape(1, -1), be2.reshape(1, -1))
    (acc3,) = agg(y3, src, dst, zf)
    (out_w,) = _epilogue(acc3, zz3, degs, False)
    return out_w[:, 0:1]


# speedup vs baseline: 11.7473x; 1.0263x over previous
import functools

import jax
import jax.numpy as jnp
from jax import lax
from jax.experimental import pallas as pl
from jax.experimental.pallas import tpu as pltpu
from jax.experimental.pallas import tpu_sc as plsc

_ROWS = 1000
_CHUNK = 80
_EPS = 1e-5


def _p_plain_body(n_nodes, h_ref, wl_ref, wr_ref, b_ref, y_ref, z_ref):
    h = h_ref[...]
    y_ref[...] = jnp.dot(h, wl_ref[...], preferred_element_type=jnp.float32)
    z_ref[...] = (jnp.dot(h, wr_ref[...], preferred_element_type=jnp.float32)
                  + b_ref[0:1, :])


def _p_bn_body(n_nodes, h_ref, wl_ref, wr_ref, b_ref, st_ref, g_ref, be_ref,
               y_ref, z_ref):
    m = st_ref[0:1, :] / n_nodes
    var = st_ref[1:2, :] / n_nodes - m * m
    scale = lax.rsqrt(var + _EPS) * g_ref[0:1, :]
    h = jnp.maximum((h_ref[...] - m) * scale + be_ref[0:1, :], 0.0)
    y_ref[...] = jnp.dot(h, wl_ref[...], preferred_element_type=jnp.float32)
    z_ref[...] = (jnp.dot(h, wr_ref[...], preferred_element_type=jnp.float32)
                  + b_ref[0:1, :])


def _prologue(h, wlT, wrT, b, stats=None, g=None, be=None):
    n, d = h.shape
    wo = wlT.shape[1]
    grid = (n // _ROWS,)
    row_spec = pl.BlockSpec((_ROWS, d), lambda i: (i, 0))
    w_spec = pl.BlockSpec((d, wo), lambda i: (0, 0))
    vec_spec = pl.BlockSpec((1, wo), lambda i: (0, 0))
    out_spec = pl.BlockSpec((_ROWS, wo), lambda i: (i, 0))
    out_shape = [jax.ShapeDtypeStruct((n, wo), jnp.float32)] * 2
    if stats is None:
        body = functools.partial(_p_plain_body, n)
        in_specs = [row_spec, w_spec, w_spec, vec_spec]
        args = (h, wlT, wrT, b)
    else:
        body = functools.partial(_p_bn_body, n)
        dvec = pl.BlockSpec((1, d), lambda i: (0, 0))
        in_specs = [row_spec, w_spec, w_spec, vec_spec,
                    pl.BlockSpec((8, d), lambda i: (0, 0)), dvec, dvec]
        args = (h, wlT, wrT, b, stats, g, be)
    return pl.pallas_call(body, grid=grid, in_specs=in_specs,
                          out_specs=[out_spec, out_spec],
                          out_shape=out_shape)(*args)


def _e_stats_body(a0_ref, a1_ref, z_ref, d0_ref, d1_ref, h_ref, st_ref):
    deg = jnp.maximum(d0_ref[...] + d1_ref[...], 1.0)
    h = (a0_ref[...] + a1_ref[...]) / deg + z_ref[...]
    h_ref[...] = h

    @pl.when(pl.program_id(0) == 0)
    def _():
        st_ref[...] = jnp.zeros_like(st_ref)

    st_ref[0:1, :] += jnp.sum(h, axis=0, keepdims=True)
    st_ref[1:2, :] += jnp.sum(h * h, axis=0, keepdims=True)


def _e_plain_body(a0_ref, a1_ref, z_ref, d0_ref, d1_ref, h_ref):
    deg = jnp.maximum(d0_ref[...] + d1_ref[...], 1.0)
    h_ref[...] = (a0_ref[...] + a1_ref[...]) / deg + z_ref[...]


def _epilogue(accs, z, degs, with_stats):
    n, wo = z.shape
    nb = n // _ROWS
    grid = (nb,)
    a0 = pl.BlockSpec((_ROWS, wo), lambda i: (i, 0))
    a1 = pl.BlockSpec((_ROWS, wo), lambda i: (i + nb, 0))
    d0 = pl.BlockSpec((_ROWS, 1), lambda i: (i, 0))
    d1 = pl.BlockSpec((_ROWS, 1), lambda i: (i + nb, 0))
    zs = pl.BlockSpec((_ROWS, wo), lambda i: (i, 0))
    out_shape = [jax.ShapeDtypeStruct((n, wo), jnp.float32)]
    out_specs = [pl.BlockSpec((_ROWS, wo), lambda i: (i, 0))]
    body = _e_stats_body if with_stats else _e_plain_body
    if with_stats:
        out_shape.append(jax.ShapeDtypeStruct((8, 128), jnp.float32))
        out_specs.append(pl.BlockSpec((8, 128), lambda i: (0, 0)))
    return pl.pallas_call(body, grid=grid,
                          in_specs=[a0, a1, zs, d0, d1],
                          out_specs=out_specs,
                          out_shape=out_shape)(accs, accs, z, degs, degs)


def _m_body(n_nodes, a0_ref, a1_ref, z_ref, d0_ref, d1_ref, wl_ref, wr_ref,
            b_ref, g_ref, be_ref, y_ref, z2_ref, h_scr, st_scr):
    p = pl.program_id(0)
    i = pl.program_id(1)

    @pl.when(p == 0)
    def _():
        deg = jnp.maximum(d0_ref[...] + d1_ref[...], 1.0)
        h = (a0_ref[...] + a1_ref[...]) / deg + z_ref[...]
        h_scr[pl.ds(i * _ROWS, _ROWS), :] = h

        @pl.when(i == 0)
        def _():
            st_scr[...] = jnp.zeros_like(st_scr)

        st_scr[0:1, :] += jnp.sum(h, axis=0, keepdims=True)
        st_scr[1:2, :] += jnp.sum(h * h, axis=0, keepdims=True)

    @pl.when(p == 1)
    def _():
        m = st_scr[0:1, :] / n_nodes
        var = st_scr[1:2, :] / n_nodes - m * m
        scale = lax.rsqrt(var + _EPS) * g_ref[0:1, :]
        h = h_scr[pl.ds(i * _ROWS, _ROWS), :]
        h = jnp.maximum((h - m) * scale + be_ref[0:1, :], 0.0)
        y_ref[...] = jnp.dot(h, wl_ref[...], preferred_element_type=jnp.float32)
        z2_ref[...] = (jnp.dot(h, wr_ref[...],
                               preferred_element_type=jnp.float32)
                       + b_ref[0:1, :])


def _mid(accs, z, degs, wlT, wrT, b, g, be):
    n, dd = z.shape
    wo = wlT.shape[1]
    nb = n // _ROWS
    grid = (2, nb)

    def once(bs):
        return pl.BlockSpec(bs, lambda p, i: (0, 0))

    def p0(bs):
        return pl.BlockSpec(bs, lambda p, i: ((1 - p) * i, 0))

    def p1(bs):
        return pl.BlockSpec(bs, lambda p, i: (p * i, 0))

    a0 = pl.BlockSpec((_ROWS, dd), lambda p, i: ((1 - p) * i, 0))
    a1 = pl.BlockSpec((_ROWS, dd), lambda p, i: ((1 - p) * i + nb, 0))
    d0 = pl.BlockSpec((_ROWS, 1), lambda p, i: ((1 - p) * i, 0))
    d1 = pl.BlockSpec((_ROWS, 1), lambda p, i: ((1 - p) * i + nb, 0))
    out_shape = [jax.ShapeDtypeStruct((n, wo), jnp.float32)] * 2
    return pl.pallas_call(
        functools.partial(_m_body, n), grid=grid,
        in_specs=[a0, a1, p0((_ROWS, dd)), d0, d1,
                  once((dd, wo)), once((dd, wo)), once((1, wo)),
                  once((1, dd)), once((1, dd))],
        out_specs=[p1((_ROWS, wo)), p1((_ROWS, wo))],
        out_shape=out_shape,
        scratch_shapes=[pltpu.VMEM((n, dd), jnp.float32),
                        pltpu.VMEM((8, dd), jnp.float32)],
    )(accs, accs, z, degs, degs, wlT, wrT, b, g, be)


def _make_sc_agg(n_nodes, width, e_total, with_deg):
    nw = 32
    epw = e_total // nw
    nch = epw // _CHUNK
    rpt = (n_nodes // 16) // 8 * 8
    tail = n_nodes - 16 * rpt
    mesh = plsc.VectorSubcoreMesh(core_axis_name="c", subcore_axis_name="s")

    nbuf = 3
    zrows = rpt // 8
    out_type = [jax.ShapeDtypeStruct((2 * n_nodes, width), jnp.float32)]
    scratch = (
        [pltpu.VMEM((_CHUNK,), jnp.int32) for _ in range(nbuf)]
        + [pltpu.VMEM((_CHUNK,), jnp.int32) for _ in range(nbuf)]
        + [pltpu.VMEM((_CHUNK, width), jnp.float32) for _ in range(nbuf)]
        + [pltpu.VMEM((zrows, width), jnp.float32)]
        + [pltpu.VMEM_SHARED((n_nodes, width), jnp.float32)]
        + [pltpu.SemaphoreType.DMA for _ in range(3 * nbuf)]
    )
    if with_deg:
        out_type.append(jax.ShapeDtypeStruct((2 * n_nodes,), jnp.float32))
        scratch += [
            pltpu.VMEM((_CHUNK,), jnp.float32),
            pltpu.VMEM((rpt,), jnp.float32),
            pltpu.VMEM_SHARED((n_nodes,), jnp.float32),
        ]

    def body(*refs):
        if with_deg:
            (y_hbm, src_hbm, dst_hbm, acc_out, deg_out) = refs[:5]
            (ones_v, zb_v, deg_sh) = refs[-3:]
            rest = refs[5:-3]
        else:
            (y_hbm, src_hbm, dst_hbm, acc_out) = refs[:4]
            rest = refs[4:]
        srcb = rest[0:nbuf]
        dstb = rest[nbuf:2 * nbuf]
        rowb = rest[2 * nbuf:3 * nbuf]
        zrow_v = rest[3 * nbuf]
        acc_sh = rest[3 * nbuf + 1]
        gsem = rest[3 * nbuf + 2:3 * nbuf + 2 + nbuf]
        isem = rest[3 * nbuf + 2 + nbuf:3 * nbuf + 2 + 2 * nbuf]
        ssem = rest[3 * nbuf + 2 + 2 * nbuf:3 * nbuf + 2 + 3 * nbuf]
        c = lax.axis_index("c")
        s = lax.axis_index("s")
        w = c * 16 + s
        r0 = s * rpt
        t0 = 16 * rpt
        zv = jnp.zeros((16,), jnp.float32)

        def fill_zr(j, carry):
            for k in range(width // 16):
                zrow_v[j, pl.ds(k * 16, 16)] = zv
            return carry

        lax.fori_loop(0, zrows, fill_zr, 0)
        for t in range(rpt // zrows):
            pltpu.sync_copy(zrow_v, acc_sh.at[pl.ds(r0 + t * zrows, zrows)])
        if with_deg:
            ov = jnp.full((16,), 1.0, jnp.float32)
            for k in range(_CHUNK // 16):
                ones_v[pl.ds(k * 16, 16)] = ov

            def fill_z(j, carry):
                zb_v[pl.ds(j * 16, 16)] = zv
                return carry

            lax.fori_loop(0, rpt // 16, fill_z, 0)
            pltpu.sync_copy(zb_v, deg_sh.at[pl.ds(r0, rpt)])
        if tail:
            @pl.when(s == 15)
            def _():
                pltpu.sync_copy(zrow_v.at[pl.ds(0, tail)],
                                acc_sh.at[pl.ds(t0, tail)])
                if with_deg:
                    pltpu.sync_copy(zb_v.at[pl.ds(0, tail)],
                                    deg_sh.at[pl.ds(t0, tail)])
        plsc.subcore_barrier()

        base0 = w * epw

        def start_idx(j, b):
            base = base0 + j * _CHUNK
            pltpu.async_copy(src_hbm.at[pl.ds(base, _CHUNK)], srcb[b], isem[b])
            pltpu.async_copy(dst_hbm.at[pl.ds(base, _CHUNK)], dstb[b], isem[b])

        def wait_idx(j, b):
            base = base0 + j * _CHUNK
            pltpu.make_async_copy(src_hbm.at[pl.ds(base, _CHUNK)], srcb[b],
                                  isem[b]).wait()
            pltpu.make_async_copy(dst_hbm.at[pl.ds(base, _CHUNK)], dstb[b],
                                  isem[b]).wait()

        def start_gather(b):
            pltpu.async_copy(y_hbm.at[srcb[b]], rowb[b], gsem[b])

        def wait_gather(b):
            pltpu.make_async_copy(y_hbm.at[srcb[b]], rowb[b], gsem[b]).wait()

        def start_scatter(b):
            pltpu.async_copy(rowb[b], acc_sh.at[dstb[b]], ssem[b], add=True)
            if with_deg:
                pltpu.async_copy(ones_v, deg_sh.at[dstb[b]], ssem[b], add=True)

        def wait_scatter(b):
            pltpu.make_async_copy(rowb[b], acc_sh.at[dstb[b]], ssem[b]).wait()
            if with_deg:
                pltpu.make_async_copy(ones_v, deg_sh.at[dstb[b]],
                                      ssem[b]).wait()

        def stage(j, b, p, q, guard):
            if guard:
                @pl.when(j >= 1)
                def _():
                    wait_scatter(q)
            else:
                if j >= 1:
                    wait_scatter(q)

            @pl.when(j + 2 < nch)
            def _():
                start_idx(j + 2, q)

            @pl.when(j + 1 < nch)
            def _():
                wait_idx(j + 1, p)
                start_gather(p)
            wait_gather(b)
            start_scatter(b)

        start_idx(0, 0)
        start_idx(1, 1)
        wait_idx(0, 0)
        start_gather(0)

        def step3(j3, carry):
            for b in range(nbuf):
                j = nbuf * j3 + b
                stage(j, b, (b + 1) % nbuf, (b + 2) % nbuf, True)
            return carry

        nfull = nch // nbuf
        lax.fori_loop(0, nfull, step3, 0)
        for j in range(nfull * nbuf, nch):
            b = j % nbuf
            stage(j, b, (j + 1) % nbuf, (j + 2) % nbuf, False)
        wait_scatter((nch - 1) % nbuf)
        plsc.subcore_barrier()
        o0 = c * n_nodes + r0
        pltpu.sync_copy(acc_sh.at[pl.ds(r0, rpt)], acc_out.at[pl.ds(o0, rpt)])
        if with_deg:
            pltpu.sync_copy(deg_sh.at[pl.ds(r0, rpt)], zb_v)
            pltpu.sync_copy(zb_v, deg_out.at[pl.ds(c * n_nodes + r0, rpt)])
        if tail:
            @pl.when(s == 15)
            def _():
                ot = c * n_nodes + t0
                pltpu.sync_copy(acc_sh.at[pl.ds(t0, tail)],
                                acc_out.at[pl.ds(ot, tail)])
                if with_deg:
                    pltpu.sync_copy(deg_sh.at[pl.ds(t0, tail)],
                                    zb_v.at[pl.ds(0, tail)])
                    pltpu.sync_copy(zb_v.at[pl.ds(0, tail)],
                                    deg_out.at[pl.ds(ot, tail)])

    return pl.kernel(body, out_type=tuple(out_type), mesh=mesh,
                     scratch_types=tuple(scratch))


def kernel(x, edge_index, Wl1, Wr1, b1, g1, be1, Wl2, Wr2, b2, g2, be2,
           Wl3, Wr3, b3):
    n, d = x.shape
    e = edge_index.shape[1]
    assert e % (32 * _CHUNK) == 0 and n % 16 == 0 and n % _ROWS == 0

    src = edge_index[0]
    dst = edge_index[1]

    agg_deg = _make_sc_agg(n, d, e, True)
    agg = _make_sc_agg(n, d, e, False)

    y1, zz1 = _prologue(x, Wl1.T, Wr1.T, b1.reshape(1, -1))
    acc1, deg_flat = agg_deg(y1, src, dst)
    degs = deg_flat.reshape(2 * n, 1)

    y2, zz2 = _mid(acc1, zz1, degs, Wl2.T, Wr2.T, b2.reshape(1, -1),
                   g1.reshape(1, -1), be1.reshape(1, -1))
    (acc2,) = agg(y2, src, dst)

    w3l = jnp.pad(Wl3.T, ((0, 0), (0, d - 1)))
    w3r = jnp.pad(Wr3.T, ((0, 0), (0, d - 1)))
    b3w = jnp.pad(b3.reshape(1, 1), ((0, 0), (0, d - 1)))
    y3, zz3 = _mid(acc2, zz2, degs, w3l, w3r, b3w,
                   g2.reshape(1, -1), be2.reshape(1, -1))
    (acc3,) = agg(y3, src, dst)
    (out_w,) = _epilogue(acc3, zz3, degs, False)
    return out_w[:, 0:1]


# speedup vs baseline: 13.1820x; 1.1221x over previous
import functools

import jax
import jax.numpy as jnp
from jax import lax
from jax.experimental import pallas as pl
from jax.experimental.pallas import tpu as pltpu
from jax.experimental.pallas import tpu_sc as plsc

_ROWS = 1000
_CHUNK = 80
_EPS = 1e-5


def _p_plain_body(n_nodes, h_ref, wl_ref, wr_ref, b_ref, y_ref, z_ref):
    h = h_ref[...]
    y_ref[...] = jnp.dot(h, wl_ref[...], preferred_element_type=jnp.float32)
    z_ref[...] = (jnp.dot(h, wr_ref[...], preferred_element_type=jnp.float32)
                  + b_ref[0:1, :])


def _p_bn_body(n_nodes, h_ref, wl_ref, wr_ref, b_ref, st_ref, g_ref, be_ref,
               y_ref, z_ref):
    m = st_ref[0:1, :] / n_nodes
    var = st_ref[1:2, :] / n_nodes - m * m
    scale = lax.rsqrt(var + _EPS) * g_ref[0:1, :]
    h = jnp.maximum((h_ref[...] - m) * scale + be_ref[0:1, :], 0.0)
    y_ref[...] = jnp.dot(h, wl_ref[...], preferred_element_type=jnp.float32)
    z_ref[...] = (jnp.dot(h, wr_ref[...], preferred_element_type=jnp.float32)
                  + b_ref[0:1, :])


def _prologue(h, wlT, wrT, b, stats=None, g=None, be=None):
    n, d = h.shape
    wo = wlT.shape[1]
    grid = (n // _ROWS,)
    row_spec = pl.BlockSpec((_ROWS, d), lambda i: (i, 0))
    w_spec = pl.BlockSpec((d, wo), lambda i: (0, 0))
    vec_spec = pl.BlockSpec((1, wo), lambda i: (0, 0))
    out_spec = pl.BlockSpec((_ROWS, wo), lambda i: (i, 0))
    out_shape = [jax.ShapeDtypeStruct((n, wo), jnp.float32)] * 2
    if stats is None:
        body = functools.partial(_p_plain_body, n)
        in_specs = [row_spec, w_spec, w_spec, vec_spec]
        args = (h, wlT, wrT, b)
    else:
        body = functools.partial(_p_bn_body, n)
        dvec = pl.BlockSpec((1, d), lambda i: (0, 0))
        in_specs = [row_spec, w_spec, w_spec, vec_spec,
                    pl.BlockSpec((8, d), lambda i: (0, 0)), dvec, dvec]
        args = (h, wlT, wrT, b, stats, g, be)
    return pl.pallas_call(body, grid=grid, in_specs=in_specs,
                          out_specs=[out_spec, out_spec],
                          out_shape=out_shape)(*args)


def _e_stats_body(a0_ref, a1_ref, z_ref, d0_ref, d1_ref, h_ref, st_ref):
    deg = jnp.maximum(d0_ref[...] + d1_ref[...], 1.0)
    h = (a0_ref[...] + a1_ref[...]) / deg + z_ref[...]
    h_ref[...] = h

    @pl.when(pl.program_id(0) == 0)
    def _():
        st_ref[...] = jnp.zeros_like(st_ref)

    st_ref[0:1, :] += jnp.sum(h, axis=0, keepdims=True)
    st_ref[1:2, :] += jnp.sum(h * h, axis=0, keepdims=True)


def _e_plain_body(a0_ref, a1_ref, z_ref, d0_ref, d1_ref, h_ref):
    deg = jnp.maximum(d0_ref[...] + d1_ref[...], 1.0)
    h_ref[...] = (a0_ref[...] + a1_ref[...]) / deg + z_ref[...]


def _epilogue(accs, z, degs, with_stats):
    n, wo = z.shape
    nb = n // _ROWS
    grid = (nb,)
    a0 = pl.BlockSpec((_ROWS, wo), lambda i: (i, 0))
    a1 = pl.BlockSpec((_ROWS, wo), lambda i: (i + nb, 0))
    d0 = pl.BlockSpec((_ROWS, 1), lambda i: (i, 0))
    d1 = pl.BlockSpec((_ROWS, 1), lambda i: (i + nb, 0))
    zs = pl.BlockSpec((_ROWS, wo), lambda i: (i, 0))
    out_shape = [jax.ShapeDtypeStruct((n, wo), jnp.float32)]
    out_specs = [pl.BlockSpec((_ROWS, wo), lambda i: (i, 0))]
    body = _e_stats_body if with_stats else _e_plain_body
    if with_stats:
        out_shape.append(jax.ShapeDtypeStruct((8, 128), jnp.float32))
        out_specs.append(pl.BlockSpec((8, 128), lambda i: (0, 0)))
    return pl.pallas_call(body, grid=grid,
                          in_specs=[a0, a1, zs, d0, d1],
                          out_specs=out_specs,
                          out_shape=out_shape)(accs, accs, z, degs, degs)


def _m_body(n_nodes, a0_ref, a1_ref, z_ref, d0_ref, d1_ref, wl_ref, wr_ref,
            b_ref, g_ref, be_ref, y_ref, z2_ref, h_scr, st_scr):
    p = pl.program_id(0)
    i = pl.program_id(1)

    @pl.when(p == 0)
    def _():
        deg = jnp.maximum(d0_ref[...] + d1_ref[...], 1.0)
        h = (a0_ref[...] + a1_ref[...]) / deg + z_ref[...]
        h_scr[pl.ds(i * _ROWS, _ROWS), :] = h

        @pl.when(i == 0)
        def _():
            st_scr[...] = jnp.zeros_like(st_scr)

        st_scr[0:1, :] += jnp.sum(h, axis=0, keepdims=True)
        st_scr[1:2, :] += jnp.sum(h * h, axis=0, keepdims=True)

    @pl.when(p == 1)
    def _():
        m = st_scr[0:1, :] / n_nodes
        var = st_scr[1:2, :] / n_nodes - m * m
        scale = lax.rsqrt(var + _EPS) * g_ref[0:1, :]
        h = h_scr[pl.ds(i * _ROWS, _ROWS), :]
        h = jnp.maximum((h - m) * scale + be_ref[0:1, :], 0.0)
        y_ref[...] = jnp.dot(h, wl_ref[...], preferred_element_type=jnp.float32)
        z2_ref[...] = (jnp.dot(h, wr_ref[...],
                               preferred_element_type=jnp.float32)
                       + b_ref[0:1, :])


def _mid(accs, z, degs, wlT, wrT, b, g, be):
    n, dd = z.shape
    wo = wlT.shape[1]
    nb = n // _ROWS
    grid = (2, nb)

    def once(bs):
        return pl.BlockSpec(bs, lambda p, i: (0, 0))

    def p0(bs):
        return pl.BlockSpec(bs, lambda p, i: ((1 - p) * i, 0))

    def p1(bs):
        return pl.BlockSpec(bs, lambda p, i: (p * i, 0))

    a0 = pl.BlockSpec((_ROWS, dd), lambda p, i: ((1 - p) * i, 0))
    a1 = pl.BlockSpec((_ROWS, dd), lambda p, i: ((1 - p) * i + nb, 0))
    d0 = pl.BlockSpec((_ROWS, 1), lambda p, i: ((1 - p) * i, 0))
    d1 = pl.BlockSpec((_ROWS, 1), lambda p, i: ((1 - p) * i + nb, 0))
    out_shape = [jax.ShapeDtypeStruct((n, wo), jnp.float32)] * 2
    return pl.pallas_call(
        functools.partial(_m_body, n), grid=grid,
        in_specs=[a0, a1, p0((_ROWS, dd)), d0, d1,
                  once((dd, wo)), once((dd, wo)), once((1, wo)),
                  once((1, dd)), once((1, dd))],
        out_specs=[p1((_ROWS, wo)), p1((_ROWS, wo))],
        out_shape=out_shape,
        scratch_shapes=[pltpu.VMEM((n, dd), jnp.float32),
                        pltpu.VMEM((8, dd), jnp.float32)],
    )(accs, accs, z, degs, degs, wlT, wrT, b, g, be)


def _make_sc_agg(n_nodes, width, e_total, with_deg, scalar=False):
    nw = 32
    epw = e_total // nw
    nch = epw // _CHUNK
    rpt = (n_nodes // 16) // 8 * 8
    tail = n_nodes - 16 * rpt
    mesh = plsc.VectorSubcoreMesh(core_axis_name="c", subcore_axis_name="s")

    nbuf = 3
    zrows = rpt // 8
    if scalar:
        out_type = [jax.ShapeDtypeStruct((2 * n_nodes,), jnp.float32)]
        row_shape, acc_shape, z_shape = (_CHUNK,), (n_nodes,), (rpt,)
    else:
        out_type = [jax.ShapeDtypeStruct((2 * n_nodes, width), jnp.float32)]
        row_shape, acc_shape, z_shape = ((_CHUNK, width), (n_nodes, width),
                                         (zrows, width))
    scratch = (
        [pltpu.VMEM((_CHUNK,), jnp.int32) for _ in range(nbuf)]
        + [pltpu.VMEM((_CHUNK,), jnp.int32) for _ in range(nbuf)]
        + [pltpu.VMEM(row_shape, jnp.float32) for _ in range(nbuf)]
        + [pltpu.VMEM(z_shape, jnp.float32)]
        + [pltpu.VMEM_SHARED(acc_shape, jnp.float32)]
        + [pltpu.SemaphoreType.DMA for _ in range(3 * nbuf)]
    )
    if with_deg:
        out_type.append(jax.ShapeDtypeStruct((2 * n_nodes,), jnp.float32))
        scratch += [
            pltpu.VMEM((_CHUNK,), jnp.float32),
            pltpu.VMEM((rpt,), jnp.float32),
            pltpu.VMEM_SHARED((n_nodes,), jnp.float32),
        ]

    def body(*refs):
        if with_deg:
            (y_hbm, src_hbm, dst_hbm, acc_out, deg_out) = refs[:5]
            (ones_v, zb_v, deg_sh) = refs[-3:]
            rest = refs[5:-3]
        else:
            (y_hbm, src_hbm, dst_hbm, acc_out) = refs[:4]
            rest = refs[4:]
        srcb = rest[0:nbuf]
        dstb = rest[nbuf:2 * nbuf]
        rowb = rest[2 * nbuf:3 * nbuf]
        zrow_v = rest[3 * nbuf]
        acc_sh = rest[3 * nbuf + 1]
        gsem = rest[3 * nbuf + 2:3 * nbuf + 2 + nbuf]
        isem = rest[3 * nbuf + 2 + nbuf:3 * nbuf + 2 + 2 * nbuf]
        ssem = rest[3 * nbuf + 2 + 2 * nbuf:3 * nbuf + 2 + 3 * nbuf]
        c = lax.axis_index("c")
        s = lax.axis_index("s")
        w = c * 16 + s
        r0 = s * rpt
        t0 = 16 * rpt
        zv = jnp.zeros((16,), jnp.float32)

        if scalar:
            def fill_zr(j, carry):
                zrow_v[pl.ds(j * 16, 16)] = zv
                return carry

            lax.fori_loop(0, rpt // 16, fill_zr, 0)
            pltpu.sync_copy(zrow_v, acc_sh.at[pl.ds(r0, rpt)])
        else:
            def fill_zr(j, carry):
                for k in range(width // 16):
                    zrow_v[j, pl.ds(k * 16, 16)] = zv
                return carry

            lax.fori_loop(0, zrows, fill_zr, 0)
            for t in range(rpt // zrows):
                pltpu.sync_copy(zrow_v,
                                acc_sh.at[pl.ds(r0 + t * zrows, zrows)])
        if with_deg:
            ov = jnp.full((16,), 1.0, jnp.float32)
            for k in range(_CHUNK // 16):
                ones_v[pl.ds(k * 16, 16)] = ov

            def fill_z(j, carry):
                zb_v[pl.ds(j * 16, 16)] = zv
                return carry

            lax.fori_loop(0, rpt // 16, fill_z, 0)
            pltpu.sync_copy(zb_v, deg_sh.at[pl.ds(r0, rpt)])
        if tail:
            @pl.when(s == 15)
            def _():
                pltpu.sync_copy(zrow_v.at[pl.ds(0, tail)],
                                acc_sh.at[pl.ds(t0, tail)])
                if with_deg:
                    pltpu.sync_copy(zb_v.at[pl.ds(0, tail)],
                                    deg_sh.at[pl.ds(t0, tail)])
        plsc.subcore_barrier()

        base0 = w * epw

        def start_idx(j, b):
            base = base0 + j * _CHUNK
            pltpu.async_copy(src_hbm.at[pl.ds(base, _CHUNK)], srcb[b], isem[b])
            pltpu.async_copy(dst_hbm.at[pl.ds(base, _CHUNK)], dstb[b], isem[b])

        def wait_idx(j, b):
            base = base0 + j * _CHUNK
            pltpu.make_async_copy(src_hbm.at[pl.ds(base, _CHUNK)], srcb[b],
                                  isem[b]).wait()
            pltpu.make_async_copy(dst_hbm.at[pl.ds(base, _CHUNK)], dstb[b],
                                  isem[b]).wait()

        def scale_idx(b):
            for k in range(_CHUNK // 16):
                v = srcb[b][pl.ds(k * 16, 16)]
                srcb[b][pl.ds(k * 16, 16)] = v * width

        def start_gather(b):
            if scalar:
                scale_idx(b)
            pltpu.async_copy(y_hbm.at[srcb[b]], rowb[b], gsem[b])

        def wait_gather(b):
            pltpu.make_async_copy(y_hbm.at[srcb[b]], rowb[b], gsem[b]).wait()

        def start_scatter(b):
            pltpu.async_copy(rowb[b], acc_sh.at[dstb[b]], ssem[b], add=True)
            if with_deg:
                pltpu.async_copy(ones_v, deg_sh.at[dstb[b]], ssem[b], add=True)

        def wait_scatter(b):
            pltpu.make_async_copy(rowb[b], acc_sh.at[dstb[b]], ssem[b]).wait()
            if with_deg:
                pltpu.make_async_copy(ones_v, deg_sh.at[dstb[b]],
                                      ssem[b]).wait()

        def stage(j, b, p, q, guard):
            if guard:
                @pl.when(j >= 1)
                def _():
                    wait_scatter(q)
            else:
                if j >= 1:
                    wait_scatter(q)

            @pl.when(j + 2 < nch)
            def _():
                start_idx(j + 2, q)

            @pl.when(j + 1 < nch)
            def _():
                wait_idx(j + 1, p)
                start_gather(p)
            wait_gather(b)
            start_scatter(b)

        start_idx(0, 0)
        start_idx(1, 1)
        wait_idx(0, 0)
        start_gather(0)

        def step3(j3, carry):
            for b in range(nbuf):
                j = nbuf * j3 + b
                stage(j, b, (b + 1) % nbuf, (b + 2) % nbuf, True)
            return carry

        nfull = nch // nbuf
        lax.fori_loop(0, nfull, step3, 0)
        for j in range(nfull * nbuf, nch):
            b = j % nbuf
            stage(j, b, (j + 1) % nbuf, (j + 2) % nbuf, False)
        wait_scatter((nch - 1) % nbuf)
        plsc.subcore_barrier()
        o0 = c * n_nodes + r0
        if scalar:
            pltpu.sync_copy(acc_sh.at[pl.ds(r0, rpt)], zrow_v)
            pltpu.sync_copy(zrow_v, acc_out.at[pl.ds(o0, rpt)])
        else:
            pltpu.sync_copy(acc_sh.at[pl.ds(r0, rpt)],
                            acc_out.at[pl.ds(o0, rpt)])
        if with_deg:
            pltpu.sync_copy(deg_sh.at[pl.ds(r0, rpt)], zb_v)
            pltpu.sync_copy(zb_v, deg_out.at[pl.ds(c * n_nodes + r0, rpt)])
        if tail:
            @pl.when(s == 15)
            def _():
                ot = c * n_nodes + t0
                if scalar:
                    pltpu.sync_copy(acc_sh.at[pl.ds(t0, tail)],
                                    zrow_v.at[pl.ds(0, tail)])
                    pltpu.sync_copy(zrow_v.at[pl.ds(0, tail)],
                                    acc_out.at[pl.ds(ot, tail)])
                else:
                    pltpu.sync_copy(acc_sh.at[pl.ds(t0, tail)],
                                    acc_out.at[pl.ds(ot, tail)])
                if with_deg:
                    pltpu.sync_copy(deg_sh.at[pl.ds(t0, tail)],
                                    zb_v.at[pl.ds(0, tail)])
                    pltpu.sync_copy(zb_v.at[pl.ds(0, tail)],
                                    deg_out.at[pl.ds(ot, tail)])

    return pl.kernel(body, out_type=tuple(out_type), mesh=mesh,
                     scratch_types=tuple(scratch))


def kernel(x, edge_index, Wl1, Wr1, b1, g1, be1, Wl2, Wr2, b2, g2, be2,
           Wl3, Wr3, b3):
    n, d = x.shape
    e = edge_index.shape[1]
    assert e % (32 * _CHUNK) == 0 and n % 16 == 0 and n % _ROWS == 0

    src = edge_index[0]
    dst = edge_index[1]

    agg_deg = _make_sc_agg(n, d, e, True)
    agg = _make_sc_agg(n, d, e, False)

    y1, zz1 = _prologue(x, Wl1.T, Wr1.T, b1.reshape(1, -1))
    acc1, deg_flat = agg_deg(y1, src, dst)
    degs = deg_flat.reshape(2 * n, 1)

    y2, zz2 = _mid(acc1, zz1, degs, Wl2.T, Wr2.T, b2.reshape(1, -1),
                   g1.reshape(1, -1), be1.reshape(1, -1))
    (acc2,) = agg(y2, src, dst)

    w3l = jnp.pad(Wl3.T, ((0, 0), (0, d - 1)))
    w3r = jnp.pad(Wr3.T, ((0, 0), (0, d - 1)))
    b3w = jnp.pad(b3.reshape(1, 1), ((0, 0), (0, d - 1)))
    y3, zz3 = _mid(acc2, zz2, degs, w3l, w3r, b3w,
                   g2.reshape(1, -1), be2.reshape(1, -1))
    agg_sc = _make_sc_agg(n, d, e, False, scalar=True)
    (acc3f,) = agg_sc(y3.reshape(-1), src, dst)
    (out_c,) = _epilogue(acc3f.reshape(2 * n, 1), zz3[:, 0:1], degs, False)
    return out_c


# speedup vs baseline: 13.1870x; 1.0004x over previous
import functools

import jax
import jax.numpy as jnp
from jax import lax
from jax.experimental import pallas as pl
from jax.experimental.pallas import tpu as pltpu
from jax.experimental.pallas import tpu_sc as plsc

_ROWS = 1000
_CHUNK = 80
_EPS = 1e-5


def _p_body(h_ref, wl_ref, wr_ref, b_ref, y_ref, z_ref):
    h = h_ref[...]
    y_ref[...] = jnp.dot(h, wl_ref[...], preferred_element_type=jnp.float32)
    z_ref[...] = (jnp.dot(h, wr_ref[...], preferred_element_type=jnp.float32)
                  + b_ref[0:1, :])


def _prologue(h, wlT, wrT, b):
    n, d = h.shape
    wo = wlT.shape[1]
    grid = (n // _ROWS,)
    row_spec = pl.BlockSpec((_ROWS, d), lambda i: (i, 0))
    w_spec = pl.BlockSpec((d, wo), lambda i: (0, 0))
    vec_spec = pl.BlockSpec((1, wo), lambda i: (0, 0))
    out_spec = pl.BlockSpec((_ROWS, wo), lambda i: (i, 0))
    out_shape = [jax.ShapeDtypeStruct((n, wo), jnp.float32)] * 2
    return pl.pallas_call(_p_body, grid=grid,
                          in_specs=[row_spec, w_spec, w_spec, vec_spec],
                          out_specs=[out_spec, out_spec],
                          out_shape=out_shape)(h, wlT, wrT, b)


def _e_body(a0_ref, a1_ref, z_ref, d0_ref, d1_ref, h_ref):
    deg = jnp.maximum(d0_ref[...] + d1_ref[...], 1.0)
    h_ref[...] = (a0_ref[...] + a1_ref[...]) / deg + z_ref[...]


def _epilogue(accs, z, degs):
    n, wo = z.shape
    nb = n // _ROWS
    grid = (nb,)
    a0 = pl.BlockSpec((_ROWS, wo), lambda i: (i, 0))
    a1 = pl.BlockSpec((_ROWS, wo), lambda i: (i + nb, 0))
    d0 = pl.BlockSpec((_ROWS, 1), lambda i: (i, 0))
    d1 = pl.BlockSpec((_ROWS, 1), lambda i: (i + nb, 0))
    zs = pl.BlockSpec((_ROWS, wo), lambda i: (i, 0))
    out_shape = [jax.ShapeDtypeStruct((n, wo), jnp.float32)]
    out_specs = [pl.BlockSpec((_ROWS, wo), lambda i: (i, 0))]
    return pl.pallas_call(_e_body, grid=grid,
                          in_specs=[a0, a1, zs, d0, d1],
                          out_specs=out_specs,
                          out_shape=out_shape)(accs, accs, z, degs, degs)


def _m_body(n_nodes, a0_ref, a1_ref, z_ref, d0_ref, d1_ref, wl_ref, wr_ref,
            b_ref, g_ref, be_ref, y_ref, z2_ref, h_scr, st_scr):
    p = pl.program_id(0)
    i = pl.program_id(1)

    @pl.when(p == 0)
    def _():
        deg = jnp.maximum(d0_ref[...] + d1_ref[...], 1.0)
        h = (a0_ref[...] + a1_ref[...]) / deg + z_ref[...]
        h_scr[pl.ds(i * _ROWS, _ROWS), :] = h

        @pl.when(i == 0)
        def _():
            st_scr[...] = jnp.zeros_like(st_scr)

        st_scr[0:1, :] += jnp.sum(h, axis=0, keepdims=True)
        st_scr[1:2, :] += jnp.sum(h * h, axis=0, keepdims=True)

    @pl.when(p == 1)
    def _():
        m = st_scr[0:1, :] / n_nodes
        var = st_scr[1:2, :] / n_nodes - m * m
        scale = lax.rsqrt(var + _EPS) * g_ref[0:1, :]
        h = h_scr[pl.ds(i * _ROWS, _ROWS), :]
        h = jnp.maximum((h - m) * scale + be_ref[0:1, :], 0.0)
        y_ref[...] = jnp.dot(h, wl_ref[...], preferred_element_type=jnp.float32)
        z2_ref[...] = (jnp.dot(h, wr_ref[...],
                               preferred_element_type=jnp.float32)
                       + b_ref[0:1, :])


def _mid(accs, z, degs, wlT, wrT, b, g, be):
    n, dd = z.shape
    wo = wlT.shape[1]
    nb = n // _ROWS
    grid = (2, nb)

    def once(bs):
        return pl.BlockSpec(bs, lambda p, i: (0, 0))

    def p0(bs):
        return pl.BlockSpec(bs, lambda p, i: ((1 - p) * i, 0))

    def p1(bs):
        return pl.BlockSpec(bs, lambda p, i: (p * i, 0))

    a0 = pl.BlockSpec((_ROWS, dd), lambda p, i: ((1 - p) * i, 0))
    a1 = pl.BlockSpec((_ROWS, dd), lambda p, i: ((1 - p) * i + nb, 0))
    d0 = pl.BlockSpec((_ROWS, 1), lambda p, i: ((1 - p) * i, 0))
    d1 = pl.BlockSpec((_ROWS, 1), lambda p, i: ((1 - p) * i + nb, 0))
    out_shape = [jax.ShapeDtypeStruct((n, wo), jnp.float32)] * 2
    return pl.pallas_call(
        functools.partial(_m_body, n), grid=grid,
        in_specs=[a0, a1, p0((_ROWS, dd)), d0, d1,
                  once((dd, wo)), once((dd, wo)), once((1, wo)),
                  once((1, dd)), once((1, dd))],
        out_specs=[p1((_ROWS, wo)), p1((_ROWS, wo))],
        out_shape=out_shape,
        scratch_shapes=[pltpu.VMEM((n, dd), jnp.float32),
                        pltpu.VMEM((8, dd), jnp.float32)],
    )(accs, accs, z, degs, degs, wlT, wrT, b, g, be)


def _make_sc_agg(n_nodes, width, e_total, with_deg, scalar=False):
    nw = 32
    epw = e_total // nw
    nch = epw // _CHUNK
    rpt = (n_nodes // 16) // 8 * 8
    tail = n_nodes - 16 * rpt
    mesh = plsc.VectorSubcoreMesh(core_axis_name="c", subcore_axis_name="s")

    nbuf = 3
    zrows = rpt // 8
    if scalar:
        out_type = [jax.ShapeDtypeStruct((2 * n_nodes,), jnp.float32)]
        row_shape, acc_shape, z_shape = (_CHUNK,), (n_nodes,), (rpt,)
    else:
        out_type = [jax.ShapeDtypeStruct((2 * n_nodes, width), jnp.float32)]
        row_shape, acc_shape, z_shape = ((_CHUNK, width), (n_nodes, width),
                                         (zrows, width))
    scratch = (
        [pltpu.VMEM((_CHUNK,), jnp.int32) for _ in range(nbuf)]
        + [pltpu.VMEM((_CHUNK,), jnp.int32) for _ in range(nbuf)]
        + [pltpu.VMEM(row_shape, jnp.float32) for _ in range(nbuf)]
        + [pltpu.VMEM(z_shape, jnp.float32)]
        + [pltpu.VMEM_SHARED(acc_shape, jnp.float32)]
        + [pltpu.SemaphoreType.DMA for _ in range(3 * nbuf)]
    )
    if with_deg:
        out_type.append(jax.ShapeDtypeStruct((2 * n_nodes,), jnp.float32))
        scratch += [
            pltpu.VMEM((_CHUNK,), jnp.float32),
            pltpu.VMEM((rpt,), jnp.float32),
            pltpu.VMEM_SHARED((n_nodes,), jnp.float32),
        ]

    def body(*refs):
        if with_deg:
            (y_hbm, src_hbm, dst_hbm, acc_out, deg_out) = refs[:5]
            (ones_v, zb_v, deg_sh) = refs[-3:]
            rest = refs[5:-3]
        else:
            (y_hbm, src_hbm, dst_hbm, acc_out) = refs[:4]
            rest = refs[4:]
        srcb = rest[0:nbuf]
        dstb = rest[nbuf:2 * nbuf]
        rowb = rest[2 * nbuf:3 * nbuf]
        zrow_v = rest[3 * nbuf]
        acc_sh = rest[3 * nbuf + 1]
        gsem = rest[3 * nbuf + 2:3 * nbuf + 2 + nbuf]
        isem = rest[3 * nbuf + 2 + nbuf:3 * nbuf + 2 + 2 * nbuf]
        ssem = rest[3 * nbuf + 2 + 2 * nbuf:3 * nbuf + 2 + 3 * nbuf]
        c = lax.axis_index("c")
        s = lax.axis_index("s")
        w = c * 16 + s
        r0 = s * rpt
        t0 = 16 * rpt
        zv = jnp.zeros((16,), jnp.float32)

        if scalar:
            def fill_zr(j, carry):
                zrow_v[pl.ds(j * 16, 16)] = zv
                return carry

            lax.fori_loop(0, rpt // 16, fill_zr, 0)
            pltpu.sync_copy(zrow_v, acc_sh.at[pl.ds(r0, rpt)])
        else:
            def fill_zr(j, carry):
                for k in range(width // 16):
                    zrow_v[j, pl.ds(k * 16, 16)] = zv
                return carry

            lax.fori_loop(0, zrows, fill_zr, 0)
            for t in range(rpt // zrows):
                pltpu.sync_copy(zrow_v,
                                acc_sh.at[pl.ds(r0 + t * zrows, zrows)])
        if with_deg:
            ov = jnp.full((16,), 1.0, jnp.float32)
            for k in range(_CHUNK // 16):
                ones_v[pl.ds(k * 16, 16)] = ov

            def fill_z(j, carry):
                zb_v[pl.ds(j * 16, 16)] = zv
                return carry

            lax.fori_loop(0, rpt // 16, fill_z, 0)
            pltpu.sync_copy(zb_v, deg_sh.at[pl.ds(r0, rpt)])
        if tail:
            @pl.when(s == 15)
            def _():
                pltpu.sync_copy(zrow_v.at[pl.ds(0, tail)],
                                acc_sh.at[pl.ds(t0, tail)])
                if with_deg:
                    pltpu.sync_copy(zb_v.at[pl.ds(0, tail)],
                                    deg_sh.at[pl.ds(t0, tail)])
        plsc.subcore_barrier()

        base0 = w * epw

        def start_idx(j, b):
            base = base0 + j * _CHUNK
            pltpu.async_copy(src_hbm.at[pl.ds(base, _CHUNK)], srcb[b], isem[b])
            pltpu.async_copy(dst_hbm.at[pl.ds(base, _CHUNK)], dstb[b], isem[b])

        def wait_idx(j, b):
            base = base0 + j * _CHUNK
            pltpu.make_async_copy(src_hbm.at[pl.ds(base, _CHUNK)], srcb[b],
                                  isem[b]).wait()
            pltpu.make_async_copy(dst_hbm.at[pl.ds(base, _CHUNK)], dstb[b],
                                  isem[b]).wait()

        def scale_idx(b):
            for k in range(_CHUNK // 16):
                v = srcb[b][pl.ds(k * 16, 16)]
                srcb[b][pl.ds(k * 16, 16)] = v * width

        def start_gather(b):
            if scalar:
                scale_idx(b)
            pltpu.async_copy(y_hbm.at[srcb[b]], rowb[b], gsem[b])

        def wait_gather(b):
            pltpu.make_async_copy(y_hbm.at[srcb[b]], rowb[b], gsem[b]).wait()

        def start_scatter(b):
            pltpu.async_copy(rowb[b], acc_sh.at[dstb[b]], ssem[b], add=True)
            if with_deg:
                pltpu.async_copy(ones_v, deg_sh.at[dstb[b]], ssem[b], add=True)

        def wait_scatter(b):
            pltpu.make_async_copy(rowb[b], acc_sh.at[dstb[b]], ssem[b]).wait()
            if with_deg:
                pltpu.make_async_copy(ones_v, deg_sh.at[dstb[b]],
                                      ssem[b]).wait()

        def stage(j, b, p, q, guard):
            if guard:
                @pl.when(j >= 1)
                def _():
                    wait_scatter(q)
            else:
                if j >= 1:
                    wait_scatter(q)

            @pl.when(j + 2 < nch)
            def _():
                start_idx(j + 2, q)

            @pl.when(j + 1 < nch)
            def _():
                wait_idx(j + 1, p)
                start_gather(p)
            wait_gather(b)
            start_scatter(b)

        start_idx(0, 0)
        start_idx(1, 1)
        wait_idx(0, 0)
        start_gather(0)

        def step3(j3, carry):
            for b in range(nbuf):
                j = nbuf * j3 + b
                stage(j, b, (b + 1) % nbuf, (b + 2) % nbuf, True)
            return carry

        nfull = nch // nbuf
        lax.fori_loop(0, nfull, step3, 0)
        for j in range(nfull * nbuf, nch):
            b = j % nbuf
            stage(j, b, (j + 1) % nbuf, (j + 2) % nbuf, False)
        wait_scatter((nch - 1) % nbuf)
        plsc.subcore_barrier()
        o0 = c * n_nodes + r0
        if scalar:
            pltpu.sync_copy(acc_sh.at[pl.ds(r0, rpt)], zrow_v)
            pltpu.sync_copy(zrow_v, acc_out.at[pl.ds(o0, rpt)])
        else:
            pltpu.sync_copy(acc_sh.at[pl.ds(r0, rpt)],
                            acc_out.at[pl.ds(o0, rpt)])
        if with_deg:
            pltpu.sync_copy(deg_sh.at[pl.ds(r0, rpt)], zb_v)
            pltpu.sync_copy(zb_v, deg_out.at[pl.ds(c * n_nodes + r0, rpt)])
        if tail:
            @pl.when(s == 15)
            def _():
                ot = c * n_nodes + t0
                if scalar:
                    pltpu.sync_copy(acc_sh.at[pl.ds(t0, tail)],
                                    zrow_v.at[pl.ds(0, tail)])
                    pltpu.sync_copy(zrow_v.at[pl.ds(0, tail)],
                                    acc_out.at[pl.ds(ot, tail)])
                else:
                    pltpu.sync_copy(acc_sh.at[pl.ds(t0, tail)],
                                    acc_out.at[pl.ds(ot, tail)])
                if with_deg:
                    pltpu.sync_copy(deg_sh.at[pl.ds(t0, tail)],
                                    zb_v.at[pl.ds(0, tail)])
                    pltpu.sync_copy(zb_v.at[pl.ds(0, tail)],
                                    deg_out.at[pl.ds(ot, tail)])

    return pl.kernel(body, out_type=tuple(out_type), mesh=mesh,
                     scratch_types=tuple(scratch))


def kernel(x, edge_index, Wl1, Wr1, b1, g1, be1, Wl2, Wr2, b2, g2, be2,
           Wl3, Wr3, b3):
    n, d = x.shape
    e = edge_index.shape[1]
    assert e % (32 * _CHUNK) == 0 and n % 16 == 0 and n % _ROWS == 0

    src = edge_index[0]
    dst = edge_index[1]

    agg_deg = _make_sc_agg(n, d, e, True)
    agg = _make_sc_agg(n, d, e, False)

    y1, zz1 = _prologue(x, Wl1.T, Wr1.T, b1.reshape(1, -1))
    acc1, deg_flat = agg_deg(y1, src, dst)
    degs = deg_flat.reshape(2 * n, 1)

    y2, zz2 = _mid(acc1, zz1, degs, Wl2.T, Wr2.T, b2.reshape(1, -1),
                   g1.reshape(1, -1), be1.reshape(1, -1))
    (acc2,) = agg(y2, src, dst)

    w3l = jnp.pad(Wl3.T, ((0, 0), (0, d - 1)))
    w3r = jnp.pad(Wr3.T, ((0, 0), (0, d - 1)))
    b3w = jnp.pad(b3.reshape(1, 1), ((0, 0), (0, d - 1)))
    y3, zz3 = _mid(acc2, zz2, degs, w3l, w3r, b3w,
                   g2.reshape(1, -1), be2.reshape(1, -1))
    agg_sc = _make_sc_agg(n, d, e, False, scalar=True)
    (acc3f,) = agg_sc(y3.reshape(-1), src, dst)
    (out_c,) = _epilogue(acc3f.reshape(2 * n, 1), zz3[:, 0:1], degs)
    return out_c
